# Initial kernel scaffold; baseline (speedup 1.0000x reference)
import functools

import jax
import jax.numpy as jnp
from jax import lax
from jax.experimental import pallas as pl
from jax.experimental.pallas import tpu as pltpu
from jax.experimental.pallas import tpu_sc as plsc

N = 10000
E = 320000
F = 128
NUM_GRAPHS = 32

NCORE = 2
NSUB = 16
NT = NCORE * NSUB
CH = 128
K = 80
E_PAD = NT * K * CH
NP = 10240
RB = 256
NBLK = NP // RB
ZCH = NP // NSUB // CH

_mesh = plsc.VectorSubcoreMesh(
    core_axis_name="c", subcore_axis_name="s", num_cores=NCORE,
    num_subcores=NSUB)


def _deg_body(dst_hbm, ones_hbm, zeros_hbm, out_hbm, idx_v, ones_v, zrow_v,
              deg_sh):
    c = lax.axis_index("c")
    s = lax.axis_index("s")
    wid = c * NSUB + s
    pltpu.sync_copy(ones_hbm, ones_v)
    pltpu.sync_copy(zeros_hbm, zrow_v)
    for j in range(ZCH):
        pltpu.sync_copy(zrow_v, deg_sh.at[pl.ds((s * ZCH + j) * CH, CH)])
    pltpu.sync_copy(dst_hbm.at[wid], idx_v)
    plsc.subcore_barrier()

    def body(g, carry):
        pltpu.sync_copy(ones_v, deg_sh.at[idx_v.at[g]], add=True)
        return carry

    lax.fori_loop(0, K, body, 0)
    plsc.subcore_barrier()
    for j in range(ZCH):
        r = (s * ZCH + j) * CH
        pltpu.sync_copy(deg_sh.at[pl.ds(r, CH)], out_hbm.at[c, pl.ds(r, CH)])


def _scatter_body(lin_hbm, src_hbm, dst_hbm, zeros_hbm, out_hbm,
                  isrc_v, idst_v, rows0, rows1, zrow_v, acc_sh, sem0, sem1):
    c = lax.axis_index("c")
    s = lax.axis_index("s")
    wid = c * NSUB + s
    pltpu.sync_copy(zeros_hbm, zrow_v)
    for j in range(ZCH):
        pltpu.sync_copy(zrow_v, acc_sh.at[pl.ds((s * ZCH + j) * CH, CH)])
    pltpu.sync_copy(src_hbm.at[wid], isrc_v)
    pltpu.sync_copy(dst_hbm.at[wid], idst_v)
    plsc.subcore_barrier()

    pltpu.async_copy(lin_hbm.at[isrc_v.at[0]], rows0, sem0)

    def body(g2, carry):
        g = g2 * 2
        pltpu.async_copy(lin_hbm.at[isrc_v.at[g + 1]], rows1, sem1)
        pltpu.make_async_copy(lin_hbm.at[isrc_v.at[g]], rows0, sem0).wait()
        pltpu.sync_copy(rows0, acc_sh.at[idst_v.at[g]], add=True)

        @pl.when(g + 2 < K)
        def _():
            pltpu.async_copy(lin_hbm.at[isrc_v.at[g + 2]], rows0, sem0)

        pltpu.make_async_copy(lin_hbm.at[isrc_v.at[g + 1]], rows1, sem1).wait()
        pltpu.sync_copy(rows1, acc_sh.at[idst_v.at[g + 1]], add=True)
        return carry

    lax.fori_loop(0, K // 2, body, 0)
    plsc.subcore_barrier()
    for j in range(ZCH):
        r = (s * ZCH + j) * CH
        pltpu.sync_copy(acc_sh.at[pl.ds(r, CH)], out_hbm.at[c, pl.ds(r, CH)])


_deg_kernel = pl.kernel(
    _deg_body,
    out_type=jax.ShapeDtypeStruct((NCORE, NP, 16), jnp.float32),
    mesh=_mesh,
    scratch_types=[
        pltpu.VMEM((K, CH), jnp.int32),
        pltpu.VMEM((CH, 16), jnp.float32),
        pltpu.VMEM((CH, 16), jnp.float32),
        pltpu.VMEM_SHARED((NP, 16), jnp.float32),
    ],
)

_scatter_kernel = pl.kernel(
    _scatter_body,
    out_type=jax.ShapeDtypeStruct((NCORE, NP, F), jnp.float32),
    mesh=_mesh,
    scratch_types=[
        pltpu.VMEM((K, CH), jnp.int32),
        pltpu.VMEM((K, CH), jnp.int32),
        pltpu.VMEM((CH, F), jnp.float32),
        pltpu.VMEM((CH, F), jnp.float32),
        pltpu.VMEM((CH, F), jnp.float32),
        pltpu.VMEM_SHARED((NP, F), jnp.float32),
        pltpu.SemaphoreType.DMA,
        pltpu.SemaphoreType.DMA,
    ],
)


def _tpre_body(x_ref, deg_ref, w_ref, lin_ref, dinv_ref):
    deg = deg_ref[0][:, 0:1] + deg_ref[1][:, 0:1] + 1.0
    dinv = jnp.broadcast_to(lax.rsqrt(deg), (RB, F))
    dinv_ref[...] = dinv
    lin_ref[...] = jnp.dot(x_ref[...], w_ref[...],
                           preferred_element_type=jnp.float32) * dinv


def _tlayer_body(s_ref, lin_ref, dinv_ref, w_ref, b_ref, out_ref):
    dinv = dinv_ref[...]
    h = jnp.maximum(dinv * (s_ref[0] + s_ref[1] + lin_ref[...]) + b_ref[...],
                    0.0)
    out_ref[...] = jnp.dot(h, w_ref[...],
                           preferred_element_type=jnp.float32) * dinv


def _tfinal_body(s_ref, lin_ref, dinv_ref, bc_ref, batch_ref,
                 wf0_ref, bf0_ref, wf1_ref, bf1_ref, wf2_ref, bf2_ref,
                 out_ref, acc_ref):
    i = pl.program_id(0)
    dinv = dinv_ref[...]
    h = jnp.maximum(
        dinv * (s_ref[0] + s_ref[1] + lin_ref[...]) + bc_ref[...], 0.0)
    gid = lax.broadcasted_iota(jnp.int32, (NUM_GRAPHS, RB), 0)
    onehot_t = (batch_ref[0] == gid).astype(jnp.float32)
    contrib = jnp.dot(onehot_t, h, preferred_element_type=jnp.float32)

    @pl.when(i == 0)
    def _():
        acc_ref[...] = contrib

    @pl.when(i > 0)
    def _():
        acc_ref[...] += contrib

    @pl.when(i == NBLK - 1)
    def _():
        o = acc_ref[...]
        o = jnp.maximum(jnp.dot(o, wf0_ref[...],
                                preferred_element_type=jnp.float32)
                        + bf0_ref[...], 0.0)
        o = jnp.maximum(jnp.dot(o, wf1_ref[...],
                                preferred_element_type=jnp.float32)
                        + bf1_ref[...], 0.0)
        o = jnp.maximum(jnp.dot(o, wf2_ref[...],
                                preferred_element_type=jnp.float32)
                        + bf2_ref[...], 0.0)
        out_ref[...] = o


_row_spec = pl.BlockSpec((RB, F), lambda i: (i, 0))
_s_spec = pl.BlockSpec((NCORE, RB, F), lambda i: (0, i, 0))
_w_spec = pl.BlockSpec((F, F), lambda i: (0, 0))
_b_spec = pl.BlockSpec((1, F), lambda i: (0, 0))

_tpre = pl.pallas_call(
    _tpre_body,
    grid=(NBLK,),
    in_specs=[
        _row_spec,
        pl.BlockSpec((NCORE, RB, 16), lambda i: (0, i, 0)),
        _w_spec,
    ],
    out_specs=[_row_spec, _row_spec],
    out_shape=[jax.ShapeDtypeStruct((NP, F), jnp.float32),
               jax.ShapeDtypeStruct((NP, F), jnp.float32)],
)

_tlayer = pl.pallas_call(
    _tlayer_body,
    grid=(NBLK,),
    in_specs=[_s_spec, _row_spec, _row_spec, _w_spec, _b_spec],
    out_specs=_row_spec,
    out_shape=jax.ShapeDtypeStruct((NP, F), jnp.float32),
)

_tfinal = pl.pallas_call(
    _tfinal_body,
    grid=(NBLK,),
    in_specs=[
        _s_spec, _row_spec, _row_spec, _b_spec,
        pl.BlockSpec((1, 1, RB), lambda i: (i, 0, 0)),
        _w_spec, _b_spec, _w_spec, _b_spec, _w_spec, _b_spec,
    ],
    out_specs=pl.BlockSpec((NUM_GRAPHS, F), lambda i: (0, 0)),
    out_shape=jax.ShapeDtypeStruct((NUM_GRAPHS, F), jnp.float32),
    scratch_shapes=[pltpu.VMEM((NUM_GRAPHS, F), jnp.float32)],
)


def kernel(x, edge_index, batch, Wc0, bc0, Wc1, bc1, Wc2, bc2,
           Wf0, bf0, Wf1, bf1, Wf2, bf2):
    x_p = jnp.pad(x, ((0, NP - N), (0, 0)))
    pad_row = jnp.int32(NP - 1)
    src_p = jnp.full((E_PAD,), pad_row, jnp.int32).at[:E].set(
        edge_index[0]).reshape(NT, K, CH)
    dst_p = jnp.full((E_PAD,), pad_row, jnp.int32).at[:E].set(
        edge_index[1]).reshape(NT, K, CH)
    batch_p = jnp.pad(batch, (0, NP - N),
                      constant_values=NUM_GRAPHS).reshape(NBLK, 1, RB)
    zeros128 = jnp.zeros((CH, F), jnp.float32)
    zeros16 = jnp.zeros((CH, 16), jnp.float32)
    ones16 = jnp.ones((CH, 16), jnp.float32)
    bc0_ = bc0.reshape(1, F)
    bc1_ = bc1.reshape(1, F)
    bc2_ = bc2.reshape(1, F)
    bf0_ = bf0.reshape(1, F)
    bf1_ = bf1.reshape(1, F)
    bf2_ = bf2.reshape(1, F)

    deg = _deg_kernel(dst_p, ones16, zeros16)

    lin0, dinv = _tpre(x_p, deg, Wc0)
    s0 = _scatter_kernel(lin0, src_p, dst_p, zeros128)
    lin1 = _tlayer(s0, lin0, dinv, Wc1, bc0_)
    s1 = _scatter_kernel(lin1, src_p, dst_p, zeros128)
    lin2 = _tlayer(s1, lin1, dinv, Wc2, bc1_)
    s2 = _scatter_kernel(lin2, src_p, dst_p, zeros128)
    out = _tfinal(s2, lin2, dinv, bc2_, batch_p,
                  Wf0, bf0_, Wf1, bf1_, Wf2, bf2_)
    return out


# speedup vs baseline: 7.5054x; 7.5054x over previous
import functools

import jax
import jax.numpy as jnp
from jax import lax
from jax.experimental import pallas as pl
from jax.experimental.pallas import tpu as pltpu
from jax.experimental.pallas import tpu_sc as plsc

N = 10000
E = 320000
F = 128
NUM_GRAPHS = 32

NCORE = 2
NSUB = 16
NT = NCORE * NSUB
CH = 128
K = 80
SB = 8
T = K // SB
E_PAD = NT * K * CH
NP = 10240
RB = 256
NBLK = NP // RB
ZCH = NP // NSUB // CH


def _deg_body(dst_hbm, ones_hbm, zeros_hbm, out_hbm, idx_v, ones_v, deg_sh):
    c = lax.axis_index("c")
    s = lax.axis_index("s")
    wid = c * NSUB + s
    pltpu.sync_copy(ones_hbm, ones_v)
    for j in range(ZCH):
        pltpu.sync_copy(zeros_hbm, deg_sh.at[pl.ds((s * ZCH + j) * CH, CH)])
    pltpu.sync_copy(dst_hbm.at[wid], idx_v)
    plsc.subcore_barrier()

    def body(g, carry):
        pltpu.sync_copy(ones_v, deg_sh.at[idx_v.at[g]], add=True)
        return carry

    lax.fori_loop(0, K, body, 0)
    plsc.subcore_barrier()
    for j in range(ZCH):
        r = (s * ZCH + j) * CH
        pltpu.sync_copy(deg_sh.at[pl.ds(r, CH)], out_hbm.at[c, pl.ds(r, CH)])


def _scatter_body(lin_hbm, src_hbm, dst_hbm, zeros_hbm, out_hbm,
                  isrc_v, idst_v, rows0, rows1, acc_sh, sem0, sem1, semi):
    c = lax.axis_index("c")
    s = lax.axis_index("s")
    wid = c * NSUB + s
    pltpu.sync_copy(zeros_hbm, rows0)
    for j in range(ZCH):
        pltpu.sync_copy(rows0, acc_sh.at[pl.ds((s * ZCH + j) * CH, CH)])
    pltpu.sync_copy(src_hbm.at[wid, 0], isrc_v.at[0])
    pltpu.sync_copy(dst_hbm.at[wid, 0], idst_v.at[0])
    pltpu.async_copy(src_hbm.at[wid, 1], isrc_v.at[1], semi)
    pltpu.async_copy(dst_hbm.at[wid, 1], idst_v.at[1], semi)
    plsc.subcore_barrier()

    pltpu.async_copy(lin_hbm.at[isrc_v.at[0, 0]], rows0, sem0)
    rows = (rows0, rows1)
    sems = (sem0, sem1)

    def outer(t, carry):
        slot = lax.rem(t, 2)
        nslot = 1 - slot
        for j in range(SB):
            cur = rows[j % 2]
            nxt = rows[(j + 1) % 2]
            if j < SB - 1:
                pltpu.async_copy(lin_hbm.at[isrc_v.at[slot, j + 1]], nxt,
                                 sems[(j + 1) % 2])
            else:
                @pl.when(t + 1 < T)
                def _():
                    pltpu.make_async_copy(
                        src_hbm.at[wid, t + 1], isrc_v.at[nslot], semi).wait()
                    pltpu.make_async_copy(
                        dst_hbm.at[wid, t + 1], idst_v.at[nslot], semi).wait()
                    pltpu.async_copy(lin_hbm.at[isrc_v.at[nslot, 0]], nxt,
                                     sems[(j + 1) % 2])
            pltpu.make_async_copy(lin_hbm.at[isrc_v.at[slot, j]], cur,
                                  sems[j % 2]).wait()
            pltpu.sync_copy(cur, acc_sh.at[idst_v.at[slot, j]], add=True)
            if j == SB - 1:
                @pl.when(t + 2 < T)
                def _():
                    pltpu.async_copy(src_hbm.at[wid, t + 2], isrc_v.at[slot],
                                     semi)
                    pltpu.async_copy(dst_hbm.at[wid, t + 2], idst_v.at[slot],
                                     semi)
        return carry

    lax.fori_loop(0, T, outer, 0)
    plsc.subcore_barrier()
    for j in range(ZCH):
        r = (s * ZCH + j) * CH
        pltpu.sync_copy(acc_sh.at[pl.ds(r, CH)], out_hbm.at[c, pl.ds(r, CH)])


@functools.lru_cache(maxsize=None)
def _sc_kernels():
    mesh = plsc.VectorSubcoreMesh(
        core_axis_name="c", subcore_axis_name="s", num_cores=NCORE,
        num_subcores=NSUB)
    deg_kernel = pl.kernel(
        _deg_body,
        out_type=jax.ShapeDtypeStruct((NCORE, NP, F), jnp.float32),
        mesh=mesh,
        scratch_types=[
            pltpu.VMEM((K, CH), jnp.int32),
            pltpu.VMEM((CH, F), jnp.float32),
            pltpu.VMEM_SHARED((NP, F), jnp.float32),
        ],
    )
    scatter_kernel = pl.kernel(
        _scatter_body,
        out_type=jax.ShapeDtypeStruct((NCORE, NP, F), jnp.float32),
        mesh=mesh,
        scratch_types=[
            pltpu.VMEM((2, SB, CH), jnp.int32),
            pltpu.VMEM((2, SB, CH), jnp.int32),
            pltpu.VMEM((CH, F), jnp.float32),
            pltpu.VMEM((CH, F), jnp.float32),
            pltpu.VMEM_SHARED((NP, F), jnp.float32),
            pltpu.SemaphoreType.DMA,
            pltpu.SemaphoreType.DMA,
            pltpu.SemaphoreType.DMA,
        ],
    )
    return deg_kernel, scatter_kernel


def _tpre_body(x_ref, deg_ref, w_ref, lin_ref, dinv_ref):
    deg = deg_ref[0][:, 0:1] + deg_ref[1][:, 0:1] + 1.0
    dinv = jnp.broadcast_to(lax.rsqrt(deg), (RB, F))
    dinv_ref[...] = dinv
    lin_ref[...] = jnp.dot(x_ref[...], w_ref[...],
                           preferred_element_type=jnp.float32) * dinv


def _tlayer_body(s_ref, lin_ref, dinv_ref, w_ref, b_ref, out_ref):
    dinv = dinv_ref[...]
    h = jnp.maximum(dinv * (s_ref[0] + s_ref[1] + lin_ref[...]) + b_ref[...],
                    0.0)
    out_ref[...] = jnp.dot(h, w_ref[...],
                           preferred_element_type=jnp.float32) * dinv


def _tfinal_body(s_ref, lin_ref, dinv_ref, bc_ref, batch_ref,
                 wf0_ref, bf0_ref, wf1_ref, bf1_ref, wf2_ref, bf2_ref,
                 out_ref, acc_ref):
    i = pl.program_id(0)
    dinv = dinv_ref[...]
    h = jnp.maximum(
        dinv * (s_ref[0] + s_ref[1] + lin_ref[...]) + bc_ref[...], 0.0)
    gid = lax.broadcasted_iota(jnp.int32, (NUM_GRAPHS, RB), 0)
    onehot_t = (batch_ref[0] == gid).astype(jnp.float32)
    contrib = jnp.dot(onehot_t, h, preferred_element_type=jnp.float32)

    @pl.when(i == 0)
    def _():
        acc_ref[...] = contrib

    @pl.when(i > 0)
    def _():
        acc_ref[...] += contrib

    @pl.when(i == NBLK - 1)
    def _():
        o = acc_ref[...]
        o = jnp.maximum(jnp.dot(o, wf0_ref[...],
                                preferred_element_type=jnp.float32)
                        + bf0_ref[...], 0.0)
        o = jnp.maximum(jnp.dot(o, wf1_ref[...],
                                preferred_element_type=jnp.float32)
                        + bf1_ref[...], 0.0)
        o = jnp.maximum(jnp.dot(o, wf2_ref[...],
                                preferred_element_type=jnp.float32)
                        + bf2_ref[...], 0.0)
        out_ref[...] = o


_row_spec = pl.BlockSpec((RB, F), lambda i: (i, 0))
_s_spec = pl.BlockSpec((NCORE, RB, F), lambda i: (0, i, 0))
_w_spec = pl.BlockSpec((F, F), lambda i: (0, 0))
_b_spec = pl.BlockSpec((1, F), lambda i: (0, 0))

_tpre = pl.pallas_call(
    _tpre_body,
    grid=(NBLK,),
    in_specs=[
        _row_spec,
        pl.BlockSpec((NCORE, RB, F), lambda i: (0, i, 0)),
        _w_spec,
    ],
    out_specs=[_row_spec, _row_spec],
    out_shape=[jax.ShapeDtypeStruct((NP, F), jnp.float32),
               jax.ShapeDtypeStruct((NP, F), jnp.float32)],
)

_tlayer = pl.pallas_call(
    _tlayer_body,
    grid=(NBLK,),
    in_specs=[_s_spec, _row_spec, _row_spec, _w_spec, _b_spec],
    out_specs=_row_spec,
    out_shape=jax.ShapeDtypeStruct((NP, F), jnp.float32),
)

_tfinal = pl.pallas_call(
    _tfinal_body,
    grid=(NBLK,),
    in_specs=[
        _s_spec, _row_spec, _row_spec, _b_spec,
        pl.BlockSpec((1, 1, RB), lambda i: (i, 0, 0)),
        _w_spec, _b_spec, _w_spec, _b_spec, _w_spec, _b_spec,
    ],
    out_specs=pl.BlockSpec((NUM_GRAPHS, F), lambda i: (0, 0)),
    out_shape=jax.ShapeDtypeStruct((NUM_GRAPHS, F), jnp.float32),
    scratch_shapes=[pltpu.VMEM((NUM_GRAPHS, F), jnp.float32)],
)


def kernel(x, edge_index, batch, Wc0, bc0, Wc1, bc1, Wc2, bc2,
           Wf0, bf0, Wf1, bf1, Wf2, bf2):
    x_p = jnp.pad(x, ((0, NP - N), (0, 0)))
    pad_row = jnp.int32(NP - 1)
    src_flat = jnp.full((E_PAD,), pad_row, jnp.int32).at[:E].set(
        edge_index[0])
    dst_flat = jnp.full((E_PAD,), pad_row, jnp.int32).at[:E].set(
        edge_index[1])
    src_p = src_flat.reshape(NT, T, SB, CH)
    dst_p = dst_flat.reshape(NT, T, SB, CH)
    dst_deg = dst_flat.reshape(NT, K, CH)
    batch_p = jnp.pad(batch, (0, NP - N),
                      constant_values=NUM_GRAPHS).reshape(NBLK, 1, RB)
    zeros128 = jnp.zeros((CH, F), jnp.float32)
    ones128 = jnp.ones((CH, F), jnp.float32)
    bc0_ = bc0.reshape(1, F)
    bc1_ = bc1.reshape(1, F)
    bc2_ = bc2.reshape(1, F)
    bf0_ = bf0.reshape(1, F)
    bf1_ = bf1.reshape(1, F)
    bf2_ = bf2.reshape(1, F)

    _deg_kernel, _scatter_kernel = _sc_kernels()
    deg = _deg_kernel(dst_deg, ones128, zeros128)

    lin0, dinv = _tpre(x_p, deg, Wc0)
    s0 = _scatter_kernel(lin0, src_p, dst_p, zeros128)
    lin1 = _tlayer(s0, lin0, dinv, Wc1, bc0_)
    s1 = _scatter_kernel(lin1, src_p, dst_p, zeros128)
    lin2 = _tlayer(s1, lin1, dinv, Wc2, bc1_)
    s2 = _scatter_kernel(lin2, src_p, dst_p, zeros128)
    out = _tfinal(s2, lin2, dinv, bc2_, batch_p,
                  Wf0, bf0_, Wf1, bf1_, Wf2, bf2_)
    return out


# speedup vs baseline: 21.9368x; 2.9228x over previous
import functools

import jax
import jax.numpy as jnp
from jax import lax
from jax.experimental import pallas as pl
from jax.experimental.pallas import tpu as pltpu
from jax.experimental.pallas import tpu_sc as plsc

N = 10000
E = 320000
F = 128
NUM_GRAPHS = 32

NCORE = 2
NSUB = 16
NT = NCORE * NSUB
CH = 128
K = 80
SB = 8
T = K // SB
E_PAD = NT * K * CH
NP = 10240
RB = 256
NBLK = NP // RB
ZCH = NP // NSUB // CH


def _deg_body(dst_hbm, ones_hbm, zeros_hbm, out_hbm, idx_v, ones_v, deg_sh):
    c = lax.axis_index("c")
    s = lax.axis_index("s")
    wid = c * NSUB + s
    pltpu.sync_copy(ones_hbm, ones_v)
    for j in range(ZCH):
        pltpu.sync_copy(zeros_hbm, deg_sh.at[pl.ds((s * ZCH + j) * CH, CH)])
    pltpu.sync_copy(dst_hbm.at[wid], idx_v)
    plsc.subcore_barrier()

    def body(g, carry):
        pltpu.sync_copy(ones_v, deg_sh.at[idx_v.at[g]], add=True)
        return carry

    lax.fori_loop(0, K, body, 0)
    plsc.subcore_barrier()
    for j in range(ZCH):
        r = (s * ZCH + j) * CH
        pltpu.sync_copy(deg_sh.at[pl.ds(r, CH)], out_hbm.at[c, pl.ds(r, CH)])


def _scatter_body(lin_hbm, src_hbm, dst_hbm, zeros_hbm, out_hbm,
                  isrc_v, idst_v, rows0, rows1, acc_sh, sem0, sem1, semi):
    c = lax.axis_index("c")
    s = lax.axis_index("s")
    wid = c * NSUB + s
    pltpu.sync_copy(zeros_hbm, rows0)
    for j in range(ZCH):
        pltpu.sync_copy(rows0, acc_sh.at[pl.ds((s * ZCH + j) * CH, CH)])
    pltpu.sync_copy(src_hbm.at[wid, 0], isrc_v.at[0])
    pltpu.sync_copy(dst_hbm.at[wid, 0], idst_v.at[0])
    pltpu.async_copy(src_hbm.at[wid, 1], isrc_v.at[1], semi)
    pltpu.async_copy(dst_hbm.at[wid, 1], idst_v.at[1], semi)
    plsc.subcore_barrier()

    pltpu.async_copy(lin_hbm.at[isrc_v.at[0, 0]], rows0, sem0)
    rows = (rows0, rows1)
    sems = (sem0, sem1)

    def outer(t, carry):
        slot = lax.rem(t, 2)
        nslot = 1 - slot
        for j in range(SB):
            cur = rows[j % 2]
            nxt = rows[(j + 1) % 2]
            if j < SB - 1:
                pltpu.async_copy(lin_hbm.at[isrc_v.at[slot, j + 1]], nxt,
                                 sems[(j + 1) % 2])
            else:
                @pl.when(t + 1 < T)
                def _():
                    pltpu.make_async_copy(
                        src_hbm.at[wid, t + 1], isrc_v.at[nslot], semi).wait()
                    pltpu.make_async_copy(
                        dst_hbm.at[wid, t + 1], idst_v.at[nslot], semi).wait()
                    pltpu.async_copy(lin_hbm.at[isrc_v.at[nslot, 0]], nxt,
                                     sems[(j + 1) % 2])
            pltpu.make_async_copy(lin_hbm.at[isrc_v.at[slot, j]], cur,
                                  sems[j % 2]).wait()
            pltpu.sync_copy(cur, acc_sh.at[idst_v.at[slot, j]], add=True)
            if j == SB - 1:
                @pl.when(t + 2 < T)
                def _():
                    pltpu.async_copy(src_hbm.at[wid, t + 2], isrc_v.at[slot],
                                     semi)
                    pltpu.async_copy(dst_hbm.at[wid, t + 2], idst_v.at[slot],
                                     semi)
        return carry

    lax.fori_loop(0, T, outer, 0)
    plsc.subcore_barrier()
    for j in range(ZCH):
        r = (s * ZCH + j) * CH
        pltpu.sync_copy(acc_sh.at[pl.ds(r, CH)], out_hbm.at[c, pl.ds(r, CH)])


@functools.lru_cache(maxsize=None)
def _sc_kernels():
    mesh = plsc.VectorSubcoreMesh(
        core_axis_name="c", subcore_axis_name="s", num_cores=NCORE,
        num_subcores=NSUB)
    deg_kernel = pl.kernel(
        _deg_body,
        out_type=jax.ShapeDtypeStruct((NCORE, NP, F), jnp.float32),
        mesh=mesh,
        scratch_types=[
            pltpu.VMEM((K, CH), jnp.int32),
            pltpu.VMEM((CH, F), jnp.float32),
            pltpu.VMEM_SHARED((NP, F), jnp.float32),
        ],
    )
    scatter_kernel = pl.kernel(
        _scatter_body,
        out_type=jax.ShapeDtypeStruct((NCORE, NP, F), jnp.float32),
        mesh=mesh,
        scratch_types=[
            pltpu.VMEM((2, SB, CH), jnp.int32),
            pltpu.VMEM((2, SB, CH), jnp.int32),
            pltpu.VMEM((CH, F), jnp.float32),
            pltpu.VMEM((CH, F), jnp.float32),
            pltpu.VMEM_SHARED((NP, F), jnp.float32),
            pltpu.SemaphoreType.DMA,
            pltpu.SemaphoreType.DMA,
            pltpu.SemaphoreType.DMA,
        ],
    )
    return deg_kernel, scatter_kernel


def _tpre_body(x_ref, deg_ref, w_ref, lin_ref, dinv_ref):
    deg = deg_ref[0][:, 0:1] + deg_ref[1][:, 0:1] + 1.0
    dinv = jnp.broadcast_to(lax.rsqrt(deg), (RB, F))
    dinv_ref[...] = dinv
    lin_ref[...] = jnp.dot(x_ref[...], w_ref[...],
                           preferred_element_type=jnp.float32) * dinv


def _tlayer_body(s_ref, lin_ref, dinv_ref, w_ref, b_ref, out_ref):
    dinv = dinv_ref[...]
    h = jnp.maximum(dinv * (s_ref[0] + s_ref[1] + lin_ref[...]) + b_ref[...],
                    0.0)
    out_ref[...] = jnp.dot(h, w_ref[...],
                           preferred_element_type=jnp.float32) * dinv


def _tfinal_body(s_ref, lin_ref, dinv_ref, bc_ref, batch_ref,
                 wf0_ref, bf0_ref, wf1_ref, bf1_ref, wf2_ref, bf2_ref,
                 out_ref, acc_ref):
    i = pl.program_id(0)
    dinv = dinv_ref[...]
    h = jnp.maximum(
        dinv * (s_ref[0] + s_ref[1] + lin_ref[...]) + bc_ref[...], 0.0)
    gid = lax.broadcasted_iota(jnp.int32, (NUM_GRAPHS, RB), 0)
    onehot_t = (batch_ref[0] == gid).astype(jnp.float32)
    contrib = jnp.dot(onehot_t, h, preferred_element_type=jnp.float32)

    @pl.when(i == 0)
    def _():
        acc_ref[...] = contrib

    @pl.when(i > 0)
    def _():
        acc_ref[...] += contrib

    @pl.when(i == NBLK - 1)
    def _():
        o = acc_ref[...]
        o = jnp.maximum(jnp.dot(o, wf0_ref[...],
                                preferred_element_type=jnp.float32)
                        + bf0_ref[...], 0.0)
        o = jnp.maximum(jnp.dot(o, wf1_ref[...],
                                preferred_element_type=jnp.float32)
                        + bf1_ref[...], 0.0)
        o = jnp.maximum(jnp.dot(o, wf2_ref[...],
                                preferred_element_type=jnp.float32)
                        + bf2_ref[...], 0.0)
        out_ref[...] = o


_row_spec = pl.BlockSpec((RB, F), lambda i: (i, 0))
_s_spec = pl.BlockSpec((NCORE, RB, F), lambda i: (0, i, 0))
_w_spec = pl.BlockSpec((F, F), lambda i: (0, 0))
_b_spec = pl.BlockSpec((1, F), lambda i: (0, 0))

_tpre = pl.pallas_call(
    _tpre_body,
    grid=(NBLK,),
    in_specs=[
        _row_spec,
        pl.BlockSpec((NCORE, RB, F), lambda i: (0, i, 0)),
        _w_spec,
    ],
    out_specs=[_row_spec, _row_spec],
    out_shape=[jax.ShapeDtypeStruct((NP, F), jnp.float32),
               jax.ShapeDtypeStruct((NP, F), jnp.float32)],
)

_tlayer = pl.pallas_call(
    _tlayer_body,
    grid=(NBLK,),
    in_specs=[_s_spec, _row_spec, _row_spec, _w_spec, _b_spec],
    out_specs=_row_spec,
    out_shape=jax.ShapeDtypeStruct((NP, F), jnp.float32),
)

_tfinal = pl.pallas_call(
    _tfinal_body,
    grid=(NBLK,),
    in_specs=[
        _s_spec, _row_spec, _row_spec, _b_spec,
        pl.BlockSpec((1, 1, RB), lambda i: (i, 0, 0)),
        _w_spec, _b_spec, _w_spec, _b_spec, _w_spec, _b_spec,
    ],
    out_specs=pl.BlockSpec((NUM_GRAPHS, F), lambda i: (0, 0)),
    out_shape=jax.ShapeDtypeStruct((NUM_GRAPHS, F), jnp.float32),
    scratch_shapes=[pltpu.VMEM((NUM_GRAPHS, F), jnp.float32)],
)


def kernel(x, edge_index, batch, Wc0, bc0, Wc1, bc1, Wc2, bc2,
           Wf0, bf0, Wf1, bf1, Wf2, bf2):
    x_p = jnp.pad(x, ((0, NP - N), (0, 0)))
    pad_rows = N + jnp.arange(E_PAD, dtype=jnp.int32) % (NP - N)
    src_flat = pad_rows.at[:E].set(edge_index[0])
    dst_flat = pad_rows.at[:E].set(edge_index[1])
    src_p = src_flat.reshape(NT, T, SB, CH)
    dst_p = dst_flat.reshape(NT, T, SB, CH)
    dst_deg = dst_flat.reshape(NT, K, CH)
    batch_p = jnp.pad(batch, (0, NP - N),
                      constant_values=NUM_GRAPHS).reshape(NBLK, 1, RB)
    zeros128 = jnp.zeros((CH, F), jnp.float32)
    ones128 = jnp.ones((CH, F), jnp.float32)
    bc0_ = bc0.reshape(1, F)
    bc1_ = bc1.reshape(1, F)
    bc2_ = bc2.reshape(1, F)
    bf0_ = bf0.reshape(1, F)
    bf1_ = bf1.reshape(1, F)
    bf2_ = bf2.reshape(1, F)

    _deg_kernel, _scatter_kernel = _sc_kernels()
    deg = _deg_kernel(dst_deg, ones128, zeros128)

    lin0, dinv = _tpre(x_p, deg, Wc0)
    s0 = _scatter_kernel(lin0, src_p, dst_p, zeros128)
    lin1 = _tlayer(s0, lin0, dinv, Wc1, bc0_)
    s1 = _scatter_kernel(lin1, src_p, dst_p, zeros128)
    lin2 = _tlayer(s1, lin1, dinv, Wc2, bc1_)
    s2 = _scatter_kernel(lin2, src_p, dst_p, zeros128)
    out = _tfinal(s2, lin2, dinv, bc2_, batch_p,
                  Wf0, bf0_, Wf1, bf1_, Wf2, bf2_)
    return out


# speedup vs baseline: 22.1925x; 1.0117x over previous
import functools

import jax
import jax.numpy as jnp
import numpy as np
from jax import lax
from jax.experimental import pallas as pl
from jax.experimental.pallas import tpu as pltpu
from jax.experimental.pallas import tpu_sc as plsc

N = 10000
E = 320000
F = 128
NUM_GRAPHS = 32

NCORE = 2
NSUB = 16
NT = NCORE * NSUB
CH = 128
K = 80
SB = 8
T = K // SB
E_PAD = NT * K * CH
NP = 10240
RB = 256
NBLK = NP // RB
ZCH = NP // NSUB // CH


def _deg_body(dst_hbm, ones_hbm, zeros_hbm, out_hbm, idx_v, ones_v, deg_sh):
    c = lax.axis_index("c")
    s = lax.axis_index("s")
    wid = c * NSUB + s
    pltpu.sync_copy(ones_hbm, ones_v)
    for j in range(ZCH):
        pltpu.sync_copy(zeros_hbm, deg_sh.at[pl.ds((s * ZCH + j) * CH, CH)])
    pltpu.sync_copy(dst_hbm.at[wid], idx_v)
    plsc.subcore_barrier()

    def body(g, carry):
        pltpu.sync_copy(ones_v, deg_sh.at[idx_v.at[g]], add=True)
        return carry

    lax.fori_loop(0, K, body, 0)
    plsc.subcore_barrier()
    for j in range(ZCH):
        r = (s * ZCH + j) * CH
        pltpu.sync_copy(deg_sh.at[pl.ds(r, CH)], out_hbm.at[c, pl.ds(r, CH)])


def _scatter_body(lin_hbm, src_hbm, dst_hbm, zeros_hbm, out_hbm,
                  isrc_v, idst_v, rows0, rows1, acc_sh, sem0, sem1, semi):
    c = lax.axis_index("c")
    s = lax.axis_index("s")
    wid = c * NSUB + s
    pltpu.sync_copy(zeros_hbm, rows0)
    for j in range(ZCH):
        pltpu.sync_copy(rows0, acc_sh.at[pl.ds((s * ZCH + j) * CH, CH)])
    pltpu.sync_copy(src_hbm.at[wid, 0], isrc_v.at[0])
    pltpu.sync_copy(dst_hbm.at[wid, 0], idst_v.at[0])
    pltpu.async_copy(src_hbm.at[wid, 1], isrc_v.at[1], semi)
    pltpu.async_copy(dst_hbm.at[wid, 1], idst_v.at[1], semi)
    plsc.subcore_barrier()

    pltpu.async_copy(lin_hbm.at[isrc_v.at[0, 0]], rows0, sem0)
    rows = (rows0, rows1)
    sems = (sem0, sem1)

    def outer(t, carry):
        slot = lax.rem(t, 2)
        nslot = 1 - slot
        for j in range(SB):
            cur = rows[j % 2]
            nxt = rows[(j + 1) % 2]
            if j < SB - 1:
                pltpu.async_copy(lin_hbm.at[isrc_v.at[slot, j + 1]], nxt,
                                 sems[(j + 1) % 2])
            else:
                @pl.when(t + 1 < T)
                def _():
                    pltpu.make_async_copy(
                        src_hbm.at[wid, t + 1], isrc_v.at[nslot], semi).wait()
                    pltpu.make_async_copy(
                        dst_hbm.at[wid, t + 1], idst_v.at[nslot], semi).wait()
                    pltpu.async_copy(lin_hbm.at[isrc_v.at[nslot, 0]], nxt,
                                     sems[(j + 1) % 2])
            pltpu.make_async_copy(lin_hbm.at[isrc_v.at[slot, j]], cur,
                                  sems[j % 2]).wait()
            pltpu.sync_copy(cur, acc_sh.at[idst_v.at[slot, j]], add=True)
            if j == SB - 1:
                @pl.when(t + 2 < T)
                def _():
                    pltpu.async_copy(src_hbm.at[wid, t + 2], isrc_v.at[slot],
                                     semi)
                    pltpu.async_copy(dst_hbm.at[wid, t + 2], idst_v.at[slot],
                                     semi)
        return carry

    lax.fori_loop(0, T, outer, 0)
    plsc.subcore_barrier()
    for j in range(ZCH):
        r = (s * ZCH + j) * CH
        pltpu.sync_copy(acc_sh.at[pl.ds(r, CH)], out_hbm.at[c, pl.ds(r, CH)])


@functools.lru_cache(maxsize=None)
def _sc_kernels():
    mesh = plsc.VectorSubcoreMesh(
        core_axis_name="c", subcore_axis_name="s", num_cores=NCORE,
        num_subcores=NSUB)
    deg_kernel = pl.kernel(
        _deg_body,
        out_type=jax.ShapeDtypeStruct((NCORE, NP, F), jnp.float32),
        mesh=mesh,
        scratch_types=[
            pltpu.VMEM((K, CH), jnp.int32),
            pltpu.VMEM((CH, F), jnp.float32),
            pltpu.VMEM_SHARED((NP, F), jnp.float32),
        ],
    )
    scatter_kernel = pl.kernel(
        _scatter_body,
        out_type=jax.ShapeDtypeStruct((NCORE, NP, F), jnp.float32),
        mesh=mesh,
        scratch_types=[
            pltpu.VMEM((2, SB, CH), jnp.int32),
            pltpu.VMEM((2, SB, CH), jnp.int32),
            pltpu.VMEM((CH, F), jnp.float32),
            pltpu.VMEM((CH, F), jnp.float32),
            pltpu.VMEM_SHARED((NP, F), jnp.float32),
            pltpu.SemaphoreType.DMA,
            pltpu.SemaphoreType.DMA,
            pltpu.SemaphoreType.DMA,
        ],
    )
    return deg_kernel, scatter_kernel


def _tpre_body(x_ref, deg_ref, w_ref, lin_ref, dinv_ref):
    deg = deg_ref[0][:, 0:1] + deg_ref[1][:, 0:1] + 1.0
    dinv_c = lax.rsqrt(deg)
    dinv = jnp.broadcast_to(dinv_c, (RB, F))
    dinv_ref[...] = jnp.broadcast_to(dinv_c, (RB, 8))
    lin_ref[...] = jnp.dot(x_ref[...], w_ref[...],
                           preferred_element_type=jnp.float32) * dinv


def _tlayer_body(s_ref, lin_ref, dinv_ref, w_ref, b_ref, out_ref):
    dinv = jnp.broadcast_to(dinv_ref[...][:, 0:1], (RB, F))
    h = jnp.maximum(dinv * (s_ref[0] + s_ref[1] + lin_ref[...]) + b_ref[...],
                    0.0)
    out_ref[...] = jnp.dot(h, w_ref[...],
                           preferred_element_type=jnp.float32) * dinv


def _tfinal_body(s_ref, lin_ref, dinv_ref, bc_ref, batch_ref,
                 wf0_ref, bf0_ref, wf1_ref, bf1_ref, wf2_ref, bf2_ref,
                 out_ref, acc_ref):
    i = pl.program_id(0)
    dinv = jnp.broadcast_to(dinv_ref[...][:, 0:1], (RB, F))
    h = jnp.maximum(
        dinv * (s_ref[0] + s_ref[1] + lin_ref[...]) + bc_ref[...], 0.0)
    gid = lax.broadcasted_iota(jnp.int32, (NUM_GRAPHS, RB), 0)
    onehot_t = (batch_ref[0] == gid).astype(jnp.float32)
    contrib = jnp.dot(onehot_t, h, preferred_element_type=jnp.float32)

    @pl.when(i == 0)
    def _():
        acc_ref[...] = contrib

    @pl.when(i > 0)
    def _():
        acc_ref[...] += contrib

    @pl.when(i == NBLK - 1)
    def _():
        o = acc_ref[...]
        o = jnp.maximum(jnp.dot(o, wf0_ref[...],
                                preferred_element_type=jnp.float32)
                        + bf0_ref[...], 0.0)
        o = jnp.maximum(jnp.dot(o, wf1_ref[...],
                                preferred_element_type=jnp.float32)
                        + bf1_ref[...], 0.0)
        o = jnp.maximum(jnp.dot(o, wf2_ref[...],
                                preferred_element_type=jnp.float32)
                        + bf2_ref[...], 0.0)
        out_ref[...] = o


_row_spec = pl.BlockSpec((RB, F), lambda i: (i, 0))
_s_spec = pl.BlockSpec((NCORE, RB, F), lambda i: (0, i, 0))
_w_spec = pl.BlockSpec((F, F), lambda i: (0, 0))
_b_spec = pl.BlockSpec((1, F), lambda i: (0, 0))
_dinv_spec = pl.BlockSpec((RB, 8), lambda i: (i, 0))

_tpre = pl.pallas_call(
    _tpre_body,
    grid=(NBLK,),
    in_specs=[
        _row_spec,
        pl.BlockSpec((NCORE, RB, F), lambda i: (0, i, 0)),
        _w_spec,
    ],
    out_specs=[_row_spec, _dinv_spec],
    out_shape=[jax.ShapeDtypeStruct((NP, F), jnp.float32),
               jax.ShapeDtypeStruct((NP, 8), jnp.float32)],
)

_tlayer = pl.pallas_call(
    _tlayer_body,
    grid=(NBLK,),
    in_specs=[_s_spec, _row_spec, _dinv_spec, _w_spec, _b_spec],
    out_specs=_row_spec,
    out_shape=jax.ShapeDtypeStruct((NP, F), jnp.float32),
)

_tfinal = pl.pallas_call(
    _tfinal_body,
    grid=(NBLK,),
    in_specs=[
        _s_spec, _row_spec, _dinv_spec, _b_spec,
        pl.BlockSpec((1, 1, RB), lambda i: (i, 0, 0)),
        _w_spec, _b_spec, _w_spec, _b_spec, _w_spec, _b_spec,
    ],
    out_specs=pl.BlockSpec((NUM_GRAPHS, F), lambda i: (0, 0)),
    out_shape=jax.ShapeDtypeStruct((NUM_GRAPHS, F), jnp.float32),
    scratch_shapes=[pltpu.VMEM((NUM_GRAPHS, F), jnp.float32)],
)


def kernel(x, edge_index, batch, Wc0, bc0, Wc1, bc1, Wc2, bc2,
           Wf0, bf0, Wf1, bf1, Wf2, bf2):
    x_p = jnp.pad(x, ((0, NP - N), (0, 0)))
    pad_tail = jnp.asarray(
        N + np.arange(E_PAD - E, dtype=np.int32) % (NP - N))
    src_flat = jnp.concatenate([edge_index[0], pad_tail])
    dst_flat = jnp.concatenate([edge_index[1], pad_tail])
    src_p = src_flat.reshape(NT, T, SB, CH)
    dst_p = dst_flat.reshape(NT, T, SB, CH)
    dst_deg = dst_flat.reshape(NT, K, CH)
    batch_p = jnp.pad(batch, (0, NP - N),
                      constant_values=NUM_GRAPHS).reshape(NBLK, 1, RB)
    zeros128 = jnp.zeros((CH, F), jnp.float32)
    ones128 = jnp.ones((CH, F), jnp.float32)
    bc0_ = bc0.reshape(1, F)
    bc1_ = bc1.reshape(1, F)
    bc2_ = bc2.reshape(1, F)
    bf0_ = bf0.reshape(1, F)
    bf1_ = bf1.reshape(1, F)
    bf2_ = bf2.reshape(1, F)

    _deg_kernel, _scatter_kernel = _sc_kernels()
    deg = _deg_kernel(dst_deg, ones128, zeros128)

    lin0, dinv = _tpre(x_p, deg, Wc0)
    s0 = _scatter_kernel(lin0, src_p, dst_p, zeros128)
    lin1 = _tlayer(s0, lin0, dinv, Wc1, bc0_)
    s1 = _scatter_kernel(lin1, src_p, dst_p, zeros128)
    lin2 = _tlayer(s1, lin1, dinv, Wc2, bc1_)
    s2 = _scatter_kernel(lin2, src_p, dst_p, zeros128)
    out = _tfinal(s2, lin2, dinv, bc2_, batch_p,
                  Wf0, bf0_, Wf1, bf1_, Wf2, bf2_)
    return out


# speedup vs baseline: 23.9638x; 1.0798x over previous
import functools

import jax
import jax.numpy as jnp
import numpy as np
from jax import lax
from jax.experimental import pallas as pl
from jax.experimental.pallas import tpu as pltpu
from jax.experimental.pallas import tpu_sc as plsc

N = 10000
E = 320000
F = 128
NUM_GRAPHS = 32

NCORE = 2
NSUB = 16
NT = NCORE * NSUB
CH = 112
K = 90
SB = 6
T = K // SB
E_PAD = NT * K * CH
NP = 10240
RB = 256
NBLK = NP // RB
ZSIZES = (112, 112, 112, 112, 112, 80)


def _deg_body(dst_hbm, ones_hbm, zeros_hbm, out_hbm, idx_v, ones_v, deg_sh,
              dsem0, dsem1):
    c = lax.axis_index("c")
    s = lax.axis_index("s")
    wid = c * NSUB + s
    pltpu.sync_copy(ones_hbm, ones_v)
    base = s * (NP // NSUB)
    off = 0
    for sz in ZSIZES:
        pltpu.sync_copy(zeros_hbm.at[pl.ds(0, sz)],
                        deg_sh.at[pl.ds(base + off, sz)])
        off += sz
    pltpu.sync_copy(dst_hbm.at[wid], idx_v)
    plsc.subcore_barrier()

    def body(m, carry):
        g = m * 2

        @pl.when(m >= 1)
        def _():
            pltpu.make_async_copy(ones_v, deg_sh.at[idx_v.at[g]],
                                  dsem0).wait()

        pltpu.async_copy(ones_v, deg_sh.at[idx_v.at[g]], dsem0, add=True)

        @pl.when(m >= 1)
        def _():
            pltpu.make_async_copy(ones_v, deg_sh.at[idx_v.at[g + 1]],
                                  dsem1).wait()

        pltpu.async_copy(ones_v, deg_sh.at[idx_v.at[g + 1]], dsem1, add=True)
        return carry

    lax.fori_loop(0, K // 2, body, 0)
    pltpu.make_async_copy(ones_v, deg_sh.at[idx_v.at[0]], dsem0).wait()
    pltpu.make_async_copy(ones_v, deg_sh.at[idx_v.at[0]], dsem1).wait()
    plsc.subcore_barrier()
    off = 0
    for sz in ZSIZES:
        r = base + off
        pltpu.sync_copy(deg_sh.at[pl.ds(r, sz)], out_hbm.at[c, pl.ds(r, sz)])
        off += sz


def _scatter_body(lin_hbm, src_hbm, dst_hbm, zeros_hbm, out_hbm,
                  isrc_v, idst_v, rb0, rb1, rb2, acc_sh,
                  g0, g1, g2, s0, s1, s2, semi):
    c = lax.axis_index("c")
    s = lax.axis_index("s")
    wid = c * NSUB + s
    pltpu.sync_copy(zeros_hbm, rb0)
    base = s * (NP // NSUB)
    off = 0
    for sz in ZSIZES:
        pltpu.sync_copy(rb0.at[pl.ds(0, sz)],
                        acc_sh.at[pl.ds(base + off, sz)])
        off += sz
    pltpu.sync_copy(src_hbm.at[wid, 0], isrc_v.at[0])
    pltpu.sync_copy(dst_hbm.at[wid, 0], idst_v.at[0])
    pltpu.async_copy(src_hbm.at[wid, 1], isrc_v.at[1], semi)
    pltpu.async_copy(dst_hbm.at[wid, 1], idst_v.at[1], semi)
    plsc.subcore_barrier()

    rbs = (rb0, rb1, rb2)
    gs = (g0, g1, g2)
    ss = (s0, s1, s2)
    pltpu.async_copy(lin_hbm.at[isrc_v.at[0, 0]], rb0, g0)
    pltpu.async_copy(lin_hbm.at[isrc_v.at[0, 1]], rb1, g1)

    def outer(t, carry):
        slot = lax.rem(t, 2)
        nslot = 1 - slot
        for j in range(SB):
            g = t * SB + j
            b = j % 3
            b2 = (j + 2) % 3
            if j == 0:
                @pl.when(g >= 1)
                def _():
                    pltpu.make_async_copy(
                        rbs[b2], acc_sh.at[idst_v.at[nslot, SB - 1]],
                        ss[b2]).wait()

                @pl.when(jnp.logical_and(t >= 1, t + 1 < T))
                def _():
                    pltpu.async_copy(src_hbm.at[wid, t + 1],
                                     isrc_v.at[nslot], semi)
                    pltpu.async_copy(dst_hbm.at[wid, t + 1],
                                     idst_v.at[nslot], semi)
            else:
                pltpu.make_async_copy(
                    rbs[b2], acc_sh.at[idst_v.at[slot, j - 1]],
                    ss[b2]).wait()
            if j < SB - 2:
                pltpu.async_copy(lin_hbm.at[isrc_v.at[slot, j + 2]],
                                 rbs[b2], gs[b2])
            else:
                @pl.when(t + 1 < T)
                def _():
                    if j == SB - 2:
                        pltpu.make_async_copy(src_hbm.at[wid, t + 1],
                                              isrc_v.at[nslot], semi).wait()
                        pltpu.make_async_copy(dst_hbm.at[wid, t + 1],
                                              idst_v.at[nslot], semi).wait()
                    pltpu.async_copy(lin_hbm.at[isrc_v.at[nslot, j + 2 - SB]],
                                     rbs[b2], gs[b2])
            pltpu.make_async_copy(lin_hbm.at[isrc_v.at[slot, j]], rbs[b],
                                  gs[b]).wait()
            pltpu.async_copy(rbs[b], acc_sh.at[idst_v.at[slot, j]], ss[b],
                             add=True)
        return carry

    lax.fori_loop(0, T, outer, 0)
    pltpu.make_async_copy(rbs[(SB - 1) % 3],
                          acc_sh.at[idst_v.at[(T - 1) % 2, SB - 1]],
                          ss[(SB - 1) % 3]).wait()
    plsc.subcore_barrier()
    off = 0
    for sz in ZSIZES:
        r = base + off
        pltpu.sync_copy(acc_sh.at[pl.ds(r, sz)], out_hbm.at[c, pl.ds(r, sz)])
        off += sz


@functools.lru_cache(maxsize=None)
def _sc_kernels():
    mesh = plsc.VectorSubcoreMesh(
        core_axis_name="c", subcore_axis_name="s", num_cores=NCORE,
        num_subcores=NSUB)
    deg_kernel = pl.kernel(
        _deg_body,
        out_type=jax.ShapeDtypeStruct((NCORE, NP, F), jnp.float32),
        mesh=mesh,
        scratch_types=[
            pltpu.VMEM((K, CH), jnp.int32),
            pltpu.VMEM((CH, F), jnp.float32),
            pltpu.VMEM_SHARED((NP, F), jnp.float32),
            pltpu.SemaphoreType.DMA,
            pltpu.SemaphoreType.DMA,
        ],
    )
    scatter_kernel = pl.kernel(
        _scatter_body,
        out_type=jax.ShapeDtypeStruct((NCORE, NP, F), jnp.float32),
        mesh=mesh,
        scratch_types=[
            pltpu.VMEM((2, SB, CH), jnp.int32),
            pltpu.VMEM((2, SB, CH), jnp.int32),
            pltpu.VMEM((CH, F), jnp.float32),
            pltpu.VMEM((CH, F), jnp.float32),
            pltpu.VMEM((CH, F), jnp.float32),
            pltpu.VMEM_SHARED((NP, F), jnp.float32),
            pltpu.SemaphoreType.DMA,
            pltpu.SemaphoreType.DMA,
            pltpu.SemaphoreType.DMA,
            pltpu.SemaphoreType.DMA,
            pltpu.SemaphoreType.DMA,
            pltpu.SemaphoreType.DMA,
            pltpu.SemaphoreType.DMA,
        ],
    )
    return deg_kernel, scatter_kernel


def _tpre_body(x_ref, deg_ref, w_ref, lin_ref, dinv_ref):
    deg = deg_ref[0][:, 0:1] + deg_ref[1][:, 0:1] + 1.0
    dinv_c = lax.rsqrt(deg)
    dinv = jnp.broadcast_to(dinv_c, (RB, F))
    dinv_ref[...] = jnp.broadcast_to(dinv_c, (RB, 8))
    lin_ref[...] = jnp.dot(x_ref[...], w_ref[...],
                           preferred_element_type=jnp.float32) * dinv


def _tlayer_body(s_ref, lin_ref, dinv_ref, w_ref, b_ref, out_ref):
    dinv = jnp.broadcast_to(dinv_ref[...][:, 0:1], (RB, F))
    h = jnp.maximum(dinv * (s_ref[0] + s_ref[1] + lin_ref[...]) + b_ref[...],
                    0.0)
    out_ref[...] = jnp.dot(h, w_ref[...],
                           preferred_element_type=jnp.float32) * dinv


def _tfinal_body(s_ref, lin_ref, dinv_ref, bc_ref, batch_ref,
                 wf0_ref, bf0_ref, wf1_ref, bf1_ref, wf2_ref, bf2_ref,
                 out_ref, acc_ref):
    i = pl.program_id(0)
    dinv = jnp.broadcast_to(dinv_ref[...][:, 0:1], (RB, F))
    h = jnp.maximum(
        dinv * (s_ref[0] + s_ref[1] + lin_ref[...]) + bc_ref[...], 0.0)
    gid = lax.broadcasted_iota(jnp.int32, (NUM_GRAPHS, RB), 0)
    onehot_t = (batch_ref[0] == gid).astype(jnp.float32)
    contrib = jnp.dot(onehot_t, h, preferred_element_type=jnp.float32)

    @pl.when(i == 0)
    def _():
        acc_ref[...] = contrib

    @pl.when(i > 0)
    def _():
        acc_ref[...] += contrib

    @pl.when(i == NBLK - 1)
    def _():
        o = acc_ref[...]
        o = jnp.maximum(jnp.dot(o, wf0_ref[...],
                                preferred_element_type=jnp.float32)
                        + bf0_ref[...], 0.0)
        o = jnp.maximum(jnp.dot(o, wf1_ref[...],
                                preferred_element_type=jnp.float32)
                        + bf1_ref[...], 0.0)
        o = jnp.maximum(jnp.dot(o, wf2_ref[...],
                                preferred_element_type=jnp.float32)
                        + bf2_ref[...], 0.0)
        out_ref[...] = o


_row_spec = pl.BlockSpec((RB, F), lambda i: (i, 0))
_s_spec = pl.BlockSpec((NCORE, RB, F), lambda i: (0, i, 0))
_w_spec = pl.BlockSpec((F, F), lambda i: (0, 0))
_b_spec = pl.BlockSpec((1, F), lambda i: (0, 0))
_dinv_spec = pl.BlockSpec((RB, 8), lambda i: (i, 0))

_tpre = pl.pallas_call(
    _tpre_body,
    grid=(NBLK,),
    in_specs=[
        _row_spec,
        pl.BlockSpec((NCORE, RB, F), lambda i: (0, i, 0)),
        _w_spec,
    ],
    out_specs=[_row_spec, _dinv_spec],
    out_shape=[jax.ShapeDtypeStruct((NP, F), jnp.float32),
               jax.ShapeDtypeStruct((NP, 8), jnp.float32)],
)

_tlayer = pl.pallas_call(
    _tlayer_body,
    grid=(NBLK,),
    in_specs=[_s_spec, _row_spec, _dinv_spec, _w_spec, _b_spec],
    out_specs=_row_spec,
    out_shape=jax.ShapeDtypeStruct((NP, F), jnp.float32),
)

_tfinal = pl.pallas_call(
    _tfinal_body,
    grid=(NBLK,),
    in_specs=[
        _s_spec, _row_spec, _dinv_spec, _b_spec,
        pl.BlockSpec((1, 1, RB), lambda i: (i, 0, 0)),
        _w_spec, _b_spec, _w_spec, _b_spec, _w_spec, _b_spec,
    ],
    out_specs=pl.BlockSpec((NUM_GRAPHS, F), lambda i: (0, 0)),
    out_shape=jax.ShapeDtypeStruct((NUM_GRAPHS, F), jnp.float32),
    scratch_shapes=[pltpu.VMEM((NUM_GRAPHS, F), jnp.float32)],
)


def kernel(x, edge_index, batch, Wc0, bc0, Wc1, bc1, Wc2, bc2,
           Wf0, bf0, Wf1, bf1, Wf2, bf2):
    x_p = jnp.pad(x, ((0, NP - N), (0, 0)))
    pad_tail = jnp.asarray(
        N + np.arange(E_PAD - E, dtype=np.int32) % (NP - N))
    src_flat = jnp.concatenate([edge_index[0], pad_tail])
    dst_flat = jnp.concatenate([edge_index[1], pad_tail])
    src_p = src_flat.reshape(NT, T, SB, CH)
    dst_p = dst_flat.reshape(NT, T, SB, CH)
    dst_deg = dst_flat.reshape(NT, K, CH)
    batch_p = jnp.pad(batch, (0, NP - N),
                      constant_values=NUM_GRAPHS).reshape(NBLK, 1, RB)
    zeros128 = jnp.zeros((CH, F), jnp.float32)
    ones128 = jnp.ones((CH, F), jnp.float32)
    bc0_ = bc0.reshape(1, F)
    bc1_ = bc1.reshape(1, F)
    bc2_ = bc2.reshape(1, F)
    bf0_ = bf0.reshape(1, F)
    bf1_ = bf1.reshape(1, F)
    bf2_ = bf2.reshape(1, F)

    _deg_kernel, _scatter_kernel = _sc_kernels()
    deg = _deg_kernel(dst_deg, ones128, zeros128)

    lin0, dinv = _tpre(x_p, deg, Wc0)
    s0 = _scatter_kernel(lin0, src_p, dst_p, zeros128)
    lin1 = _tlayer(s0, lin0, dinv, Wc1, bc0_)
    s1 = _scatter_kernel(lin1, src_p, dst_p, zeros128)
    lin2 = _tlayer(s1, lin1, dinv, Wc2, bc1_)
    s2 = _scatter_kernel(lin2, src_p, dst_p, zeros128)
    out = _tfinal(s2, lin2, dinv, bc2_, batch_p,
                  Wf0, bf0_, Wf1, bf1_, Wf2, bf2_)
    return out


# speedup vs baseline: 26.0928x; 1.0888x over previous
import functools

import jax
import jax.numpy as jnp
import numpy as np
from jax import lax
from jax.experimental import pallas as pl
from jax.experimental.pallas import tpu as pltpu
from jax.experimental.pallas import tpu_sc as plsc

N = 10000
E = 320000
F = 128
NUM_GRAPHS = 32

NCORE = 2
NSUB = 16
NT = NCORE * NSUB
CH = 112
K = 90
SB = 6
T = K // SB
E_PAD = NT * K * CH
NP = 10240
RB = 512
NBLK = NP // RB
ZSIZES = (112, 112, 112, 112, 112, 80)


def _deg_body(dst_hbm, ones_hbm, zeros_hbm, out_hbm, idx_v, ones_v, deg_sh,
              dsem0, dsem1):
    c = lax.axis_index("c")
    s = lax.axis_index("s")
    wid = c * NSUB + s
    pltpu.sync_copy(ones_hbm, ones_v)
    base = s * (NP // NSUB)
    off = 0
    for sz in ZSIZES:
        pltpu.sync_copy(zeros_hbm.at[pl.ds(0, sz)],
                        deg_sh.at[pl.ds(base + off, sz)])
        off += sz
    pltpu.sync_copy(dst_hbm.at[wid], idx_v)
    plsc.subcore_barrier()

    def body(m, carry):
        g = m * 2

        @pl.when(m >= 1)
        def _():
            pltpu.make_async_copy(ones_v, deg_sh.at[idx_v.at[g]],
                                  dsem0).wait()

        pltpu.async_copy(ones_v, deg_sh.at[idx_v.at[g]], dsem0, add=True)

        @pl.when(m >= 1)
        def _():
            pltpu.make_async_copy(ones_v, deg_sh.at[idx_v.at[g + 1]],
                                  dsem1).wait()

        pltpu.async_copy(ones_v, deg_sh.at[idx_v.at[g + 1]], dsem1, add=True)
        return carry

    lax.fori_loop(0, K // 2, body, 0)
    pltpu.make_async_copy(ones_v, deg_sh.at[idx_v.at[0]], dsem0).wait()
    pltpu.make_async_copy(ones_v, deg_sh.at[idx_v.at[0]], dsem1).wait()
    plsc.subcore_barrier()
    off = 0
    for sz in ZSIZES:
        r = base + off
        pltpu.sync_copy(deg_sh.at[pl.ds(r, sz)], out_hbm.at[c, pl.ds(r, sz)])
        off += sz


def _scatter_body(lin_hbm, src_hbm, dst_hbm, zeros_hbm, out_hbm,
                  isrc_v, idst_v, rb0, rb1, rb2, acc_sh,
                  g0, g1, g2, s0, s1, s2, semi):
    c = lax.axis_index("c")
    s = lax.axis_index("s")
    wid = c * NSUB + s
    pltpu.sync_copy(zeros_hbm, rb0)
    base = s * (NP // NSUB)
    off = 0
    for sz in ZSIZES:
        pltpu.sync_copy(rb0.at[pl.ds(0, sz)],
                        acc_sh.at[pl.ds(base + off, sz)])
        off += sz
    pltpu.sync_copy(src_hbm.at[wid, 0], isrc_v.at[0])
    pltpu.sync_copy(dst_hbm.at[wid, 0], idst_v.at[0])
    pltpu.async_copy(src_hbm.at[wid, 1], isrc_v.at[1], semi)
    pltpu.async_copy(dst_hbm.at[wid, 1], idst_v.at[1], semi)
    plsc.subcore_barrier()

    rbs = (rb0, rb1, rb2)
    gs = (g0, g1, g2)
    ss = (s0, s1, s2)
    pltpu.async_copy(lin_hbm.at[isrc_v.at[0, 0]], rb0, g0)
    pltpu.async_copy(lin_hbm.at[isrc_v.at[0, 1]], rb1, g1)

    def outer(t, carry):
        slot = lax.rem(t, 2)
        nslot = 1 - slot
        for j in range(SB):
            g = t * SB + j
            b = j % 3
            b2 = (j + 2) % 3
            if j == 0:
                @pl.when(g >= 1)
                def _():
                    pltpu.make_async_copy(
                        rbs[b2], acc_sh.at[idst_v.at[nslot, SB - 1]],
                        ss[b2]).wait()

                @pl.when(jnp.logical_and(t >= 1, t + 1 < T))
                def _():
                    pltpu.async_copy(src_hbm.at[wid, t + 1],
                                     isrc_v.at[nslot], semi)
                    pltpu.async_copy(dst_hbm.at[wid, t + 1],
                                     idst_v.at[nslot], semi)
            else:
                pltpu.make_async_copy(
                    rbs[b2], acc_sh.at[idst_v.at[slot, j - 1]],
                    ss[b2]).wait()
            if j < SB - 2:
                pltpu.async_copy(lin_hbm.at[isrc_v.at[slot, j + 2]],
                                 rbs[b2], gs[b2])
            else:
                @pl.when(t + 1 < T)
                def _():
                    if j == SB - 2:
                        pltpu.make_async_copy(src_hbm.at[wid, t + 1],
                                              isrc_v.at[nslot], semi).wait()
                        pltpu.make_async_copy(dst_hbm.at[wid, t + 1],
                                              idst_v.at[nslot], semi).wait()
                    pltpu.async_copy(lin_hbm.at[isrc_v.at[nslot, j + 2 - SB]],
                                     rbs[b2], gs[b2])
            pltpu.make_async_copy(lin_hbm.at[isrc_v.at[slot, j]], rbs[b],
                                  gs[b]).wait()
            pltpu.async_copy(rbs[b], acc_sh.at[idst_v.at[slot, j]], ss[b],
                             add=True)
        return carry

    lax.fori_loop(0, T, outer, 0)
    pltpu.make_async_copy(rbs[(SB - 1) % 3],
                          acc_sh.at[idst_v.at[(T - 1) % 2, SB - 1]],
                          ss[(SB - 1) % 3]).wait()
    plsc.subcore_barrier()
    off = 0
    for sz in ZSIZES:
        r = base + off
        pltpu.sync_copy(acc_sh.at[pl.ds(r, sz)], out_hbm.at[c, pl.ds(r, sz)])
        off += sz


@functools.lru_cache(maxsize=None)
def _sc_kernels():
    mesh = plsc.VectorSubcoreMesh(
        core_axis_name="c", subcore_axis_name="s", num_cores=NCORE,
        num_subcores=NSUB)
    deg_kernel = pl.kernel(
        _deg_body,
        out_type=jax.ShapeDtypeStruct((NCORE, NP, F), jnp.float32),
        mesh=mesh,
        scratch_types=[
            pltpu.VMEM((K, CH), jnp.int32),
            pltpu.VMEM((CH, F), jnp.float32),
            pltpu.VMEM_SHARED((NP, F), jnp.float32),
            pltpu.SemaphoreType.DMA,
            pltpu.SemaphoreType.DMA,
        ],
    )
    scatter_kernel = pl.kernel(
        _scatter_body,
        out_type=jax.ShapeDtypeStruct((NCORE, NP, F), jnp.float32),
        mesh=mesh,
        scratch_types=[
            pltpu.VMEM((2, SB, CH), jnp.int32),
            pltpu.VMEM((2, SB, CH), jnp.int32),
            pltpu.VMEM((CH, F), jnp.float32),
            pltpu.VMEM((CH, F), jnp.float32),
            pltpu.VMEM((CH, F), jnp.float32),
            pltpu.VMEM_SHARED((NP, F), jnp.float32),
            pltpu.SemaphoreType.DMA,
            pltpu.SemaphoreType.DMA,
            pltpu.SemaphoreType.DMA,
            pltpu.SemaphoreType.DMA,
            pltpu.SemaphoreType.DMA,
            pltpu.SemaphoreType.DMA,
            pltpu.SemaphoreType.DMA,
        ],
    )
    return deg_kernel, scatter_kernel


def _tpre_body(x_ref, deg_ref, w_ref, lin_ref, dinv_ref):
    deg = deg_ref[0][:, 0:1] + deg_ref[1][:, 0:1] + 1.0
    dinv_c = lax.rsqrt(deg)
    dinv = jnp.broadcast_to(dinv_c, (RB, F))
    dinv_ref[...] = jnp.broadcast_to(dinv_c, (RB, 8))
    lin_ref[...] = jnp.dot(x_ref[...], w_ref[...],
                           preferred_element_type=jnp.float32) * dinv


def _tlayer_body(s_ref, lin_ref, dinv_ref, w_ref, b_ref, out_ref):
    dinv = jnp.broadcast_to(dinv_ref[...][:, 0:1], (RB, F))
    h = jnp.maximum(dinv * (s_ref[0] + s_ref[1] + lin_ref[...]) + b_ref[...],
                    0.0)
    out_ref[...] = jnp.dot(h, w_ref[...],
                           preferred_element_type=jnp.float32) * dinv


def _tfinal_body(s_ref, lin_ref, dinv_ref, bc_ref, batch_ref,
                 wf0_ref, bf0_ref, wf1_ref, bf1_ref, wf2_ref, bf2_ref,
                 out_ref, acc_ref):
    i = pl.program_id(0)
    dinv = jnp.broadcast_to(dinv_ref[...][:, 0:1], (RB, F))
    h = jnp.maximum(
        dinv * (s_ref[0] + s_ref[1] + lin_ref[...]) + bc_ref[...], 0.0)
    gid = lax.broadcasted_iota(jnp.int32, (NUM_GRAPHS, RB), 0)
    onehot_t = (batch_ref[0] == gid).astype(jnp.float32)
    contrib = jnp.dot(onehot_t, h, preferred_element_type=jnp.float32)

    @pl.when(i == 0)
    def _():
        acc_ref[...] = contrib

    @pl.when(i > 0)
    def _():
        acc_ref[...] += contrib

    @pl.when(i == NBLK - 1)
    def _():
        o = acc_ref[...]
        o = jnp.maximum(jnp.dot(o, wf0_ref[...],
                                preferred_element_type=jnp.float32)
                        + bf0_ref[...], 0.0)
        o = jnp.maximum(jnp.dot(o, wf1_ref[...],
                                preferred_element_type=jnp.float32)
                        + bf1_ref[...], 0.0)
        o = jnp.maximum(jnp.dot(o, wf2_ref[...],
                                preferred_element_type=jnp.float32)
                        + bf2_ref[...], 0.0)
        out_ref[...] = o


_row_spec = pl.BlockSpec((RB, F), lambda i: (i, 0))
_s_spec = pl.BlockSpec((NCORE, RB, F), lambda i: (0, i, 0))
_w_spec = pl.BlockSpec((F, F), lambda i: (0, 0))
_b_spec = pl.BlockSpec((1, F), lambda i: (0, 0))
_dinv_spec = pl.BlockSpec((RB, 8), lambda i: (i, 0))

_tpre = pl.pallas_call(
    _tpre_body,
    grid=(NBLK,),
    in_specs=[
        _row_spec,
        pl.BlockSpec((NCORE, RB, F), lambda i: (0, i, 0)),
        _w_spec,
    ],
    out_specs=[_row_spec, _dinv_spec],
    out_shape=[jax.ShapeDtypeStruct((NP, F), jnp.float32),
               jax.ShapeDtypeStruct((NP, 8), jnp.float32)],
)

_tlayer = pl.pallas_call(
    _tlayer_body,
    grid=(NBLK,),
    in_specs=[_s_spec, _row_spec, _dinv_spec, _w_spec, _b_spec],
    out_specs=_row_spec,
    out_shape=jax.ShapeDtypeStruct((NP, F), jnp.float32),
)

_tfinal = pl.pallas_call(
    _tfinal_body,
    grid=(NBLK,),
    in_specs=[
        _s_spec, _row_spec, _dinv_spec, _b_spec,
        pl.BlockSpec((1, 1, RB), lambda i: (i, 0, 0)),
        _w_spec, _b_spec, _w_spec, _b_spec, _w_spec, _b_spec,
    ],
    out_specs=pl.BlockSpec((NUM_GRAPHS, F), lambda i: (0, 0)),
    out_shape=jax.ShapeDtypeStruct((NUM_GRAPHS, F), jnp.float32),
    scratch_shapes=[pltpu.VMEM((NUM_GRAPHS, F), jnp.float32)],
)


def kernel(x, edge_index, batch, Wc0, bc0, Wc1, bc1, Wc2, bc2,
           Wf0, bf0, Wf1, bf1, Wf2, bf2):
    x_p = jnp.pad(x, ((0, NP - N), (0, 0)))
    pad_tail = jnp.asarray(
        N + np.arange(E_PAD - E, dtype=np.int32) % (NP - N))
    src_flat = jnp.concatenate([edge_index[0], pad_tail])
    dst_flat = jnp.concatenate([edge_index[1], pad_tail])
    src_p = src_flat.reshape(NT, T, SB, CH)
    dst_p = dst_flat.reshape(NT, T, SB, CH)
    dst_deg = dst_flat.reshape(NT, K, CH)
    batch_p = jnp.pad(batch, (0, NP - N),
                      constant_values=NUM_GRAPHS).reshape(NBLK, 1, RB)
    zeros128 = jnp.zeros((CH, F), jnp.float32)
    ones128 = jnp.ones((CH, F), jnp.float32)
    bc0_ = bc0.reshape(1, F)
    bc1_ = bc1.reshape(1, F)
    bc2_ = bc2.reshape(1, F)
    bf0_ = bf0.reshape(1, F)
    bf1_ = bf1.reshape(1, F)
    bf2_ = bf2.reshape(1, F)

    _deg_kernel, _scatter_kernel = _sc_kernels()
    deg = _deg_kernel(dst_deg, ones128, zeros128)

    lin0, dinv = _tpre(x_p, deg, Wc0)
    s0 = _scatter_kernel(lin0, src_p, dst_p, zeros128)
    lin1 = _tlayer(s0, lin0, dinv, Wc1, bc0_)
    s1 = _scatter_kernel(lin1, src_p, dst_p, zeros128)
    lin2 = _tlayer(s1, lin1, dinv, Wc2, bc1_)
    s2 = _scatter_kernel(lin2, src_p, dst_p, zeros128)
    out = _tfinal(s2, lin2, dinv, bc2_, batch_p,
                  Wf0, bf0_, Wf1, bf1_, Wf2, bf2_)
    return out


# speedup vs baseline: 27.3665x; 1.0488x over previous
import functools

import jax
import jax.numpy as jnp
import numpy as np
from jax import lax
from jax.experimental import pallas as pl
from jax.experimental.pallas import tpu as pltpu
from jax.experimental.pallas import tpu_sc as plsc

N = 10000
E = 320000
F = 128
NUM_GRAPHS = 32

NCORE = 2
NSUB = 16
NT = NCORE * NSUB
CH = 112
K = 90
SB = 6
T = K // SB
E_PAD = NT * K * CH
NP = 10240
RB = 1024
NBLK = NP // RB
ZSIZES = (112, 112, 112, 112, 112, 80)


def _deg_body(dst_hbm, ones_hbm, zeros_hbm, out_hbm, idx_v, ones_v, deg_sh,
              dsem0, dsem1):
    c = lax.axis_index("c")
    s = lax.axis_index("s")
    wid = c * NSUB + s
    pltpu.sync_copy(ones_hbm, ones_v)
    base = s * (NP // NSUB)
    off = 0
    for sz in ZSIZES:
        pltpu.sync_copy(zeros_hbm.at[pl.ds(0, sz)],
                        deg_sh.at[pl.ds(base + off, sz)])
        off += sz
    pltpu.sync_copy(dst_hbm.at[wid], idx_v)
    plsc.subcore_barrier()

    def body(m, carry):
        g = m * 2

        @pl.when(m >= 1)
        def _():
            pltpu.make_async_copy(ones_v, deg_sh.at[idx_v.at[g]],
                                  dsem0).wait()

        pltpu.async_copy(ones_v, deg_sh.at[idx_v.at[g]], dsem0, add=True)

        @pl.when(m >= 1)
        def _():
            pltpu.make_async_copy(ones_v, deg_sh.at[idx_v.at[g + 1]],
                                  dsem1).wait()

        pltpu.async_copy(ones_v, deg_sh.at[idx_v.at[g + 1]], dsem1, add=True)
        return carry

    lax.fori_loop(0, K // 2, body, 0)
    pltpu.make_async_copy(ones_v, deg_sh.at[idx_v.at[0]], dsem0).wait()
    pltpu.make_async_copy(ones_v, deg_sh.at[idx_v.at[0]], dsem1).wait()
    plsc.subcore_barrier()
    off = 0
    for sz in ZSIZES:
        r = base + off
        pltpu.sync_copy(deg_sh.at[pl.ds(r, sz)], out_hbm.at[c, pl.ds(r, sz)])
        off += sz


def _scatter_body(lin_hbm, src_hbm, dst_hbm, zeros_hbm, out_hbm,
                  isrc_v, idst_v, rb0, rb1, rb2, acc_sh,
                  g0, g1, g2, s0, s1, s2, semi):
    c = lax.axis_index("c")
    s = lax.axis_index("s")
    wid = c * NSUB + s
    pltpu.sync_copy(zeros_hbm, rb0)
    base = s * (NP // NSUB)
    off = 0
    for sz in ZSIZES:
        pltpu.sync_copy(rb0.at[pl.ds(0, sz)],
                        acc_sh.at[pl.ds(base + off, sz)])
        off += sz
    pltpu.sync_copy(src_hbm.at[wid, 0], isrc_v.at[0])
    pltpu.sync_copy(dst_hbm.at[wid, 0], idst_v.at[0])
    pltpu.async_copy(src_hbm.at[wid, 1], isrc_v.at[1], semi)
    pltpu.async_copy(dst_hbm.at[wid, 1], idst_v.at[1], semi)
    plsc.subcore_barrier()

    rbs = (rb0, rb1, rb2)
    gs = (g0, g1, g2)
    ss = (s0, s1, s2)
    pltpu.async_copy(lin_hbm.at[isrc_v.at[0, 0]], rb0, g0)
    pltpu.async_copy(lin_hbm.at[isrc_v.at[0, 1]], rb1, g1)

    def outer(t, carry):
        slot = lax.rem(t, 2)
        nslot = 1 - slot
        for j in range(SB):
            g = t * SB + j
            b = j % 3
            b2 = (j + 2) % 3
            if j == 0:
                @pl.when(g >= 1)
                def _():
                    pltpu.make_async_copy(
                        rbs[b2], acc_sh.at[idst_v.at[nslot, SB - 1]],
                        ss[b2]).wait()

                @pl.when(jnp.logical_and(t >= 1, t + 1 < T))
                def _():
                    pltpu.async_copy(src_hbm.at[wid, t + 1],
                                     isrc_v.at[nslot], semi)
                    pltpu.async_copy(dst_hbm.at[wid, t + 1],
                                     idst_v.at[nslot], semi)
            else:
                pltpu.make_async_copy(
                    rbs[b2], acc_sh.at[idst_v.at[slot, j - 1]],
                    ss[b2]).wait()
            if j < SB - 2:
                pltpu.async_copy(lin_hbm.at[isrc_v.at[slot, j + 2]],
                                 rbs[b2], gs[b2])
            else:
                @pl.when(t + 1 < T)
                def _():
                    if j == SB - 2:
                        pltpu.make_async_copy(src_hbm.at[wid, t + 1],
                                              isrc_v.at[nslot], semi).wait()
                        pltpu.make_async_copy(dst_hbm.at[wid, t + 1],
                                              idst_v.at[nslot], semi).wait()
                    pltpu.async_copy(lin_hbm.at[isrc_v.at[nslot, j + 2 - SB]],
                                     rbs[b2], gs[b2])
            pltpu.make_async_copy(lin_hbm.at[isrc_v.at[slot, j]], rbs[b],
                                  gs[b]).wait()
            pltpu.async_copy(rbs[b], acc_sh.at[idst_v.at[slot, j]], ss[b],
                             add=True)
        return carry

    lax.fori_loop(0, T, outer, 0)
    pltpu.make_async_copy(rbs[(SB - 1) % 3],
                          acc_sh.at[idst_v.at[(T - 1) % 2, SB - 1]],
                          ss[(SB - 1) % 3]).wait()
    plsc.subcore_barrier()
    off = 0
    for sz in ZSIZES:
        r = base + off
        pltpu.sync_copy(acc_sh.at[pl.ds(r, sz)], out_hbm.at[c, pl.ds(r, sz)])
        off += sz


@functools.lru_cache(maxsize=None)
def _sc_kernels():
    mesh = plsc.VectorSubcoreMesh(
        core_axis_name="c", subcore_axis_name="s", num_cores=NCORE,
        num_subcores=NSUB)
    deg_kernel = pl.kernel(
        _deg_body,
        out_type=jax.ShapeDtypeStruct((NCORE, NP, F), jnp.float32),
        mesh=mesh,
        scratch_types=[
            pltpu.VMEM((K, CH), jnp.int32),
            pltpu.VMEM((CH, F), jnp.float32),
            pltpu.VMEM_SHARED((NP, F), jnp.float32),
            pltpu.SemaphoreType.DMA,
            pltpu.SemaphoreType.DMA,
        ],
    )
    scatter_kernel = pl.kernel(
        _scatter_body,
        out_type=jax.ShapeDtypeStruct((NCORE, NP, F), jnp.float32),
        mesh=mesh,
        scratch_types=[
            pltpu.VMEM((2, SB, CH), jnp.int32),
            pltpu.VMEM((2, SB, CH), jnp.int32),
            pltpu.VMEM((CH, F), jnp.float32),
            pltpu.VMEM((CH, F), jnp.float32),
            pltpu.VMEM((CH, F), jnp.float32),
            pltpu.VMEM_SHARED((NP, F), jnp.float32),
            pltpu.SemaphoreType.DMA,
            pltpu.SemaphoreType.DMA,
            pltpu.SemaphoreType.DMA,
            pltpu.SemaphoreType.DMA,
            pltpu.SemaphoreType.DMA,
            pltpu.SemaphoreType.DMA,
            pltpu.SemaphoreType.DMA,
        ],
    )
    return deg_kernel, scatter_kernel


def _tpre_body(x_ref, deg_ref, w_ref, lin_ref, dinv_ref):
    deg = deg_ref[0][:, 0:1] + deg_ref[1][:, 0:1] + 1.0
    dinv_c = lax.rsqrt(deg)
    dinv = jnp.broadcast_to(dinv_c, (RB, F))
    dinv_ref[...] = jnp.broadcast_to(dinv_c, (RB, 8))
    lin_ref[...] = jnp.dot(x_ref[...], w_ref[...],
                           preferred_element_type=jnp.float32) * dinv


def _tlayer_body(s_ref, lin_ref, dinv_ref, w_ref, b_ref, out_ref):
    dinv = jnp.broadcast_to(dinv_ref[...][:, 0:1], (RB, F))
    h = jnp.maximum(dinv * (s_ref[0] + s_ref[1] + lin_ref[...]) + b_ref[...],
                    0.0)
    out_ref[...] = jnp.dot(h, w_ref[...],
                           preferred_element_type=jnp.float32) * dinv


def _tfinal_body(s_ref, lin_ref, dinv_ref, bc_ref, batch_ref,
                 wf0_ref, bf0_ref, wf1_ref, bf1_ref, wf2_ref, bf2_ref,
                 out_ref, acc_ref):
    i = pl.program_id(0)
    dinv = jnp.broadcast_to(dinv_ref[...][:, 0:1], (RB, F))
    h = jnp.maximum(
        dinv * (s_ref[0] + s_ref[1] + lin_ref[...]) + bc_ref[...], 0.0)
    gid = lax.broadcasted_iota(jnp.int32, (NUM_GRAPHS, RB), 0)
    onehot_t = (batch_ref[0] == gid).astype(jnp.float32)
    contrib = jnp.dot(onehot_t, h, preferred_element_type=jnp.float32)

    @pl.when(i == 0)
    def _():
        acc_ref[...] = contrib

    @pl.when(i > 0)
    def _():
        acc_ref[...] += contrib

    @pl.when(i == NBLK - 1)
    def _():
        o = acc_ref[...]
        o = jnp.maximum(jnp.dot(o, wf0_ref[...],
                                preferred_element_type=jnp.float32)
                        + bf0_ref[...], 0.0)
        o = jnp.maximum(jnp.dot(o, wf1_ref[...],
                                preferred_element_type=jnp.float32)
                        + bf1_ref[...], 0.0)
        o = jnp.maximum(jnp.dot(o, wf2_ref[...],
                                preferred_element_type=jnp.float32)
                        + bf2_ref[...], 0.0)
        out_ref[...] = o


_row_spec = pl.BlockSpec((RB, F), lambda i: (i, 0))
_s_spec = pl.BlockSpec((NCORE, RB, F), lambda i: (0, i, 0))
_w_spec = pl.BlockSpec((F, F), lambda i: (0, 0))
_b_spec = pl.BlockSpec((1, F), lambda i: (0, 0))
_dinv_spec = pl.BlockSpec((RB, 8), lambda i: (i, 0))

_tpre = pl.pallas_call(
    _tpre_body,
    grid=(NBLK,),
    in_specs=[
        _row_spec,
        pl.BlockSpec((NCORE, RB, F), lambda i: (0, i, 0)),
        _w_spec,
    ],
    out_specs=[_row_spec, _dinv_spec],
    out_shape=[jax.ShapeDtypeStruct((NP, F), jnp.float32),
               jax.ShapeDtypeStruct((NP, 8), jnp.float32)],
)

_tlayer = pl.pallas_call(
    _tlayer_body,
    grid=(NBLK,),
    in_specs=[_s_spec, _row_spec, _dinv_spec, _w_spec, _b_spec],
    out_specs=_row_spec,
    out_shape=jax.ShapeDtypeStruct((NP, F), jnp.float32),
)

_tfinal = pl.pallas_call(
    _tfinal_body,
    grid=(NBLK,),
    in_specs=[
        _s_spec, _row_spec, _dinv_spec, _b_spec,
        pl.BlockSpec((1, 1, RB), lambda i: (i, 0, 0)),
        _w_spec, _b_spec, _w_spec, _b_spec, _w_spec, _b_spec,
    ],
    out_specs=pl.BlockSpec((NUM_GRAPHS, F), lambda i: (0, 0)),
    out_shape=jax.ShapeDtypeStruct((NUM_GRAPHS, F), jnp.float32),
    scratch_shapes=[pltpu.VMEM((NUM_GRAPHS, F), jnp.float32)],
)


def kernel(x, edge_index, batch, Wc0, bc0, Wc1, bc1, Wc2, bc2,
           Wf0, bf0, Wf1, bf1, Wf2, bf2):
    x_p = jnp.pad(x, ((0, NP - N), (0, 0)))
    pad_tail = jnp.asarray(
        N + np.arange(E_PAD - E, dtype=np.int32) % (NP - N))
    src_flat = jnp.concatenate([edge_index[0], pad_tail])
    dst_flat = jnp.concatenate([edge_index[1], pad_tail])
    src_p = src_flat.reshape(NT, T, SB, CH)
    dst_p = dst_flat.reshape(NT, T, SB, CH)
    dst_deg = dst_flat.reshape(NT, K, CH)
    batch_p = jnp.pad(batch, (0, NP - N),
                      constant_values=NUM_GRAPHS).reshape(NBLK, 1, RB)
    zeros128 = jnp.zeros((CH, F), jnp.float32)
    ones128 = jnp.ones((CH, F), jnp.float32)
    bc0_ = bc0.reshape(1, F)
    bc1_ = bc1.reshape(1, F)
    bc2_ = bc2.reshape(1, F)
    bf0_ = bf0.reshape(1, F)
    bf1_ = bf1.reshape(1, F)
    bf2_ = bf2.reshape(1, F)

    _deg_kernel, _scatter_kernel = _sc_kernels()
    deg = _deg_kernel(dst_deg, ones128, zeros128)

    lin0, dinv = _tpre(x_p, deg, Wc0)
    s0 = _scatter_kernel(lin0, src_p, dst_p, zeros128)
    lin1 = _tlayer(s0, lin0, dinv, Wc1, bc0_)
    s1 = _scatter_kernel(lin1, src_p, dst_p, zeros128)
    lin2 = _tlayer(s1, lin1, dinv, Wc2, bc1_)
    s2 = _scatter_kernel(lin2, src_p, dst_p, zeros128)
    out = _tfinal(s2, lin2, dinv, bc2_, batch_p,
                  Wf0, bf0_, Wf1, bf1_, Wf2, bf2_)
    return out


# speedup vs baseline: 27.9282x; 1.0205x over previous
import functools

import jax
import jax.numpy as jnp
import numpy as np
from jax import lax
from jax.experimental import pallas as pl
from jax.experimental.pallas import tpu as pltpu
from jax.experimental.pallas import tpu_sc as plsc

N = 10000
E = 320000
F = 128
NUM_GRAPHS = 32

NCORE = 2
NSUB = 16
NT = NCORE * NSUB
CH = 112
K = 90
SB = 6
T = K // SB
E_PAD = NT * K * CH
NP = 10240
RB = 2048
NBLK = NP // RB
ZSIZES = (112, 112, 112, 112, 112, 80)


def _deg_body(dst_hbm, ones_hbm, zeros_hbm, out_hbm, idx_v, ones_v, deg_sh,
              dsem0, dsem1):
    c = lax.axis_index("c")
    s = lax.axis_index("s")
    wid = c * NSUB + s
    pltpu.sync_copy(ones_hbm, ones_v)
    base = s * (NP // NSUB)
    off = 0
    for sz in ZSIZES:
        pltpu.sync_copy(zeros_hbm.at[pl.ds(0, sz)],
                        deg_sh.at[pl.ds(base + off, sz)])
        off += sz
    pltpu.sync_copy(dst_hbm.at[wid], idx_v)
    plsc.subcore_barrier()

    def body(m, carry):
        g = m * 2

        @pl.when(m >= 1)
        def _():
            pltpu.make_async_copy(ones_v, deg_sh.at[idx_v.at[g]],
                                  dsem0).wait()

        pltpu.async_copy(ones_v, deg_sh.at[idx_v.at[g]], dsem0, add=True)

        @pl.when(m >= 1)
        def _():
            pltpu.make_async_copy(ones_v, deg_sh.at[idx_v.at[g + 1]],
                                  dsem1).wait()

        pltpu.async_copy(ones_v, deg_sh.at[idx_v.at[g + 1]], dsem1, add=True)
        return carry

    lax.fori_loop(0, K // 2, body, 0)
    pltpu.make_async_copy(ones_v, deg_sh.at[idx_v.at[0]], dsem0).wait()
    pltpu.make_async_copy(ones_v, deg_sh.at[idx_v.at[0]], dsem1).wait()
    plsc.subcore_barrier()
    off = 0
    for sz in ZSIZES:
        r = base + off
        pltpu.sync_copy(deg_sh.at[pl.ds(r, sz)], out_hbm.at[c, pl.ds(r, sz)])
        off += sz


def _scatter_body(lin_hbm, src_hbm, dst_hbm, zeros_hbm, out_hbm,
                  isrc_v, idst_v, rb0, rb1, rb2, acc_sh,
                  g0, g1, g2, s0, s1, s2, semi):
    c = lax.axis_index("c")
    s = lax.axis_index("s")
    wid = c * NSUB + s
    pltpu.sync_copy(zeros_hbm, rb0)
    base = s * (NP // NSUB)
    off = 0
    for sz in ZSIZES:
        pltpu.sync_copy(rb0.at[pl.ds(0, sz)],
                        acc_sh.at[pl.ds(base + off, sz)])
        off += sz
    pltpu.sync_copy(src_hbm.at[wid, 0], isrc_v.at[0])
    pltpu.sync_copy(dst_hbm.at[wid, 0], idst_v.at[0])
    pltpu.async_copy(src_hbm.at[wid, 1], isrc_v.at[1], semi)
    pltpu.async_copy(dst_hbm.at[wid, 1], idst_v.at[1], semi)
    plsc.subcore_barrier()

    rbs = (rb0, rb1, rb2)
    gs = (g0, g1, g2)
    ss = (s0, s1, s2)
    pltpu.async_copy(lin_hbm.at[isrc_v.at[0, 0]], rb0, g0)
    pltpu.async_copy(lin_hbm.at[isrc_v.at[0, 1]], rb1, g1)

    def outer(t, carry):
        slot = lax.rem(t, 2)
        nslot = 1 - slot
        for j in range(SB):
            g = t * SB + j
            b = j % 3
            b2 = (j + 2) % 3
            if j == 0:
                @pl.when(g >= 1)
                def _():
                    pltpu.make_async_copy(
                        rbs[b2], acc_sh.at[idst_v.at[nslot, SB - 1]],
                        ss[b2]).wait()

                @pl.when(jnp.logical_and(t >= 1, t + 1 < T))
                def _():
                    pltpu.async_copy(src_hbm.at[wid, t + 1],
                                     isrc_v.at[nslot], semi)
                    pltpu.async_copy(dst_hbm.at[wid, t + 1],
                                     idst_v.at[nslot], semi)
            else:
                pltpu.make_async_copy(
                    rbs[b2], acc_sh.at[idst_v.at[slot, j - 1]],
                    ss[b2]).wait()
            if j < SB - 2:
                pltpu.async_copy(lin_hbm.at[isrc_v.at[slot, j + 2]],
                                 rbs[b2], gs[b2])
            else:
                @pl.when(t + 1 < T)
                def _():
                    if j == SB - 2:
                        pltpu.make_async_copy(src_hbm.at[wid, t + 1],
                                              isrc_v.at[nslot], semi).wait()
                        pltpu.make_async_copy(dst_hbm.at[wid, t + 1],
                                              idst_v.at[nslot], semi).wait()
                    pltpu.async_copy(lin_hbm.at[isrc_v.at[nslot, j + 2 - SB]],
                                     rbs[b2], gs[b2])
            pltpu.make_async_copy(lin_hbm.at[isrc_v.at[slot, j]], rbs[b],
                                  gs[b]).wait()
            pltpu.async_copy(rbs[b], acc_sh.at[idst_v.at[slot, j]], ss[b],
                             add=True)
        return carry

    lax.fori_loop(0, T, outer, 0)
    pltpu.make_async_copy(rbs[(SB - 1) % 3],
                          acc_sh.at[idst_v.at[(T - 1) % 2, SB - 1]],
                          ss[(SB - 1) % 3]).wait()
    plsc.subcore_barrier()
    off = 0
    for sz in ZSIZES:
        r = base + off
        pltpu.sync_copy(acc_sh.at[pl.ds(r, sz)], out_hbm.at[c, pl.ds(r, sz)])
        off += sz


@functools.lru_cache(maxsize=None)
def _sc_kernels():
    mesh = plsc.VectorSubcoreMesh(
        core_axis_name="c", subcore_axis_name="s", num_cores=NCORE,
        num_subcores=NSUB)
    deg_kernel = pl.kernel(
        _deg_body,
        out_type=jax.ShapeDtypeStruct((NCORE, NP, F), jnp.float32),
        mesh=mesh,
        scratch_types=[
            pltpu.VMEM((K, CH), jnp.int32),
            pltpu.VMEM((CH, F), jnp.float32),
            pltpu.VMEM_SHARED((NP, F), jnp.float32),
            pltpu.SemaphoreType.DMA,
            pltpu.SemaphoreType.DMA,
        ],
    )
    scatter_kernel = pl.kernel(
        _scatter_body,
        out_type=jax.ShapeDtypeStruct((NCORE, NP, F), jnp.float32),
        mesh=mesh,
        scratch_types=[
            pltpu.VMEM((2, SB, CH), jnp.int32),
            pltpu.VMEM((2, SB, CH), jnp.int32),
            pltpu.VMEM((CH, F), jnp.float32),
            pltpu.VMEM((CH, F), jnp.float32),
            pltpu.VMEM((CH, F), jnp.float32),
            pltpu.VMEM_SHARED((NP, F), jnp.float32),
            pltpu.SemaphoreType.DMA,
            pltpu.SemaphoreType.DMA,
            pltpu.SemaphoreType.DMA,
            pltpu.SemaphoreType.DMA,
            pltpu.SemaphoreType.DMA,
            pltpu.SemaphoreType.DMA,
            pltpu.SemaphoreType.DMA,
        ],
    )
    return deg_kernel, scatter_kernel


def _tpre_body(x_ref, deg_ref, w_ref, lin_ref, dinv_ref):
    deg = deg_ref[0][:, 0:1] + deg_ref[1][:, 0:1] + 1.0
    dinv_c = lax.rsqrt(deg)
    dinv = jnp.broadcast_to(dinv_c, (RB, F))
    dinv_ref[...] = jnp.broadcast_to(dinv_c, (RB, 8))
    lin_ref[...] = jnp.dot(x_ref[...], w_ref[...],
                           preferred_element_type=jnp.float32) * dinv


def _tlayer_body(s_ref, lin_ref, dinv_ref, w_ref, b_ref, out_ref):
    dinv = jnp.broadcast_to(dinv_ref[...][:, 0:1], (RB, F))
    h = jnp.maximum(dinv * (s_ref[0] + s_ref[1] + lin_ref[...]) + b_ref[...],
                    0.0)
    out_ref[...] = jnp.dot(h, w_ref[...],
                           preferred_element_type=jnp.float32) * dinv


def _tfinal_body(s_ref, lin_ref, dinv_ref, bc_ref, batch_ref,
                 wf0_ref, bf0_ref, wf1_ref, bf1_ref, wf2_ref, bf2_ref,
                 out_ref, acc_ref):
    i = pl.program_id(0)
    dinv = jnp.broadcast_to(dinv_ref[...][:, 0:1], (RB, F))
    h = jnp.maximum(
        dinv * (s_ref[0] + s_ref[1] + lin_ref[...]) + bc_ref[...], 0.0)
    gid = lax.broadcasted_iota(jnp.int32, (NUM_GRAPHS, RB), 0)
    onehot_t = (batch_ref[0] == gid).astype(jnp.float32)
    contrib = jnp.dot(onehot_t, h, preferred_element_type=jnp.float32)

    @pl.when(i == 0)
    def _():
        acc_ref[...] = contrib

    @pl.when(i > 0)
    def _():
        acc_ref[...] += contrib

    @pl.when(i == NBLK - 1)
    def _():
        o = acc_ref[...]
        o = jnp.maximum(jnp.dot(o, wf0_ref[...],
                                preferred_element_type=jnp.float32)
                        + bf0_ref[...], 0.0)
        o = jnp.maximum(jnp.dot(o, wf1_ref[...],
                                preferred_element_type=jnp.float32)
                        + bf1_ref[...], 0.0)
        o = jnp.maximum(jnp.dot(o, wf2_ref[...],
                                preferred_element_type=jnp.float32)
                        + bf2_ref[...], 0.0)
        out_ref[...] = o


_row_spec = pl.BlockSpec((RB, F), lambda i: (i, 0))
_s_spec = pl.BlockSpec((NCORE, RB, F), lambda i: (0, i, 0))
_w_spec = pl.BlockSpec((F, F), lambda i: (0, 0))
_b_spec = pl.BlockSpec((1, F), lambda i: (0, 0))
_dinv_spec = pl.BlockSpec((RB, 8), lambda i: (i, 0))

_tpre = pl.pallas_call(
    _tpre_body,
    grid=(NBLK,),
    in_specs=[
        _row_spec,
        pl.BlockSpec((NCORE, RB, F), lambda i: (0, i, 0)),
        _w_spec,
    ],
    out_specs=[_row_spec, _dinv_spec],
    out_shape=[jax.ShapeDtypeStruct((NP, F), jnp.float32),
               jax.ShapeDtypeStruct((NP, 8), jnp.float32)],
)

_tlayer = pl.pallas_call(
    _tlayer_body,
    grid=(NBLK,),
    in_specs=[_s_spec, _row_spec, _dinv_spec, _w_spec, _b_spec],
    out_specs=_row_spec,
    out_shape=jax.ShapeDtypeStruct((NP, F), jnp.float32),
)

_tfinal = pl.pallas_call(
    _tfinal_body,
    grid=(NBLK,),
    in_specs=[
        _s_spec, _row_spec, _dinv_spec, _b_spec,
        pl.BlockSpec((1, 1, RB), lambda i: (i, 0, 0)),
        _w_spec, _b_spec, _w_spec, _b_spec, _w_spec, _b_spec,
    ],
    out_specs=pl.BlockSpec((NUM_GRAPHS, F), lambda i: (0, 0)),
    out_shape=jax.ShapeDtypeStruct((NUM_GRAPHS, F), jnp.float32),
    scratch_shapes=[pltpu.VMEM((NUM_GRAPHS, F), jnp.float32)],
)


def kernel(x, edge_index, batch, Wc0, bc0, Wc1, bc1, Wc2, bc2,
           Wf0, bf0, Wf1, bf1, Wf2, bf2):
    x_p = jnp.pad(x, ((0, NP - N), (0, 0)))
    pad_tail = jnp.asarray(
        N + np.arange(E_PAD - E, dtype=np.int32) % (NP - N))
    src_flat = jnp.concatenate([edge_index[0], pad_tail])
    dst_flat = jnp.concatenate([edge_index[1], pad_tail])
    src_p = src_flat.reshape(NT, T, SB, CH)
    dst_p = dst_flat.reshape(NT, T, SB, CH)
    dst_deg = dst_flat.reshape(NT, K, CH)
    batch_p = jnp.pad(batch, (0, NP - N),
                      constant_values=NUM_GRAPHS).reshape(NBLK, 1, RB)
    zeros128 = jnp.zeros((CH, F), jnp.float32)
    ones128 = jnp.ones((CH, F), jnp.float32)
    bc0_ = bc0.reshape(1, F)
    bc1_ = bc1.reshape(1, F)
    bc2_ = bc2.reshape(1, F)
    bf0_ = bf0.reshape(1, F)
    bf1_ = bf1.reshape(1, F)
    bf2_ = bf2.reshape(1, F)

    _deg_kernel, _scatter_kernel = _sc_kernels()
    deg = _deg_kernel(dst_deg, ones128, zeros128)

    lin0, dinv = _tpre(x_p, deg, Wc0)
    s0 = _scatter_kernel(lin0, src_p, dst_p, zeros128)
    lin1 = _tlayer(s0, lin0, dinv, Wc1, bc0_)
    s1 = _scatter_kernel(lin1, src_p, dst_p, zeros128)
    lin2 = _tlayer(s1, lin1, dinv, Wc2, bc1_)
    s2 = _scatter_kernel(lin2, src_p, dst_p, zeros128)
    out = _tfinal(s2, lin2, dinv, bc2_, batch_p,
                  Wf0, bf0_, Wf1, bf1_, Wf2, bf2_)
    return out
